# Initial kernel scaffold; baseline (speedup 1.0000x reference)
import jax
import jax.numpy as jnp
from jax.experimental import pallas as pl


def kernel(x_state, x_goal, ei_ss, ei_gg, ei_sg, ei_gs, batch_state, batch_goal, depth, W_gcn_s_0, b_gcn_s_0, W_gcn_g_0, b_gcn_g_0, Wl_sg_0, bl_sg_0, Wr_sg_0, Wl_gs_0, bl_gs_0, Wr_gs_0, W_gcn_s_1, b_gcn_s_1, W_gcn_g_1, b_gcn_g_1, Wl_sg_1, bl_sg_1, Wr_sg_1, Wl_gs_1, bl_gs_1, Wr_gs_1, W_out, b_out):
    raise NotImplementedError("write your pallas kernel here")


# speedup vs baseline: 2.4130x; 2.4130x over previous
import jax
import jax.numpy as jnp
from jax.experimental import pallas as pl

N = 10000
B = 256
H = 64


def _proj_body(z_ref, w_ref, b_ref, o_ref):
    o_ref[...] = z_ref[...] @ w_ref[...] + b_ref[...]


def kernel(x_state, x_goal, ei_ss, ei_gg, ei_sg, ei_gs, batch_state, batch_goal, depth,
           W_gcn_s_0, b_gcn_s_0, W_gcn_g_0, b_gcn_g_0, Wl_sg_0, bl_sg_0, Wr_sg_0,
           Wl_gs_0, bl_gs_0, Wr_gs_0,
           W_gcn_s_1, b_gcn_s_1, W_gcn_g_1, b_gcn_g_1, Wl_sg_1, bl_sg_1, Wr_sg_1,
           Wl_gs_1, bl_gs_1, Wr_gs_1, W_out, b_out):
    ei_ss = ei_ss.astype(jnp.int32)
    ei_gg = ei_gg.astype(jnp.int32)
    ei_sg = ei_sg.astype(jnp.int32)
    ei_gs = ei_gs.astype(jnp.int32)

    f32 = jnp.float32
    deg_ss = jnp.zeros((N,), f32).at[ei_ss[1]].add(1.0) + 1.0
    deg_gg = jnp.zeros((N,), f32).at[ei_gg[1]].add(1.0) + 1.0
    dinv_ss = jax.lax.rsqrt(deg_ss)[:, None]
    dinv_gg = jax.lax.rsqrt(deg_gg)[:, None]
    cnt_gs = jnp.maximum(jnp.zeros((N,), f32).at[ei_gs[1]].add(1.0), 1.0)[:, None]
    cnt_sg = jnp.maximum(jnp.zeros((N,), f32).at[ei_sg[1]].add(1.0), 1.0)[:, None]

    layers = [
        (W_gcn_s_0, b_gcn_s_0, W_gcn_g_0, b_gcn_g_0, Wl_sg_0, bl_sg_0, Wr_sg_0, Wl_gs_0, bl_gs_0, Wr_gs_0),
        (W_gcn_s_1, b_gcn_s_1, W_gcn_g_1, b_gcn_g_1, Wl_sg_1, bl_sg_1, Wr_sg_1, Wl_gs_1, bl_gs_1, Wr_gs_1),
    ]
    xs, xg = x_state, x_goal
    for (Wgs, bgs, Wgg, bgg, Wlsg, blsg, Wrsg, Wlgs, blgs, Wrgs) in layers:
        ps = xs @ jnp.concatenate([Wgs, Wrgs, Wlsg], axis=1)
        pg = xg @ jnp.concatenate([Wgg, Wrsg, Wlgs], axis=1)
        us = dinv_ss * ps[:, :H]
        ug = dinv_gg * pg[:, :H]
        agg_ss = jnp.zeros((N, H), f32).at[ei_ss[1]].add(us[ei_ss[0]])
        agg_gg = jnp.zeros((N, H), f32).at[ei_gg[1]].add(ug[ei_gg[0]])
        agg_gs = jnp.zeros((N, H), f32).at[ei_gs[1]].add(pg[:, 2 * H:][ei_gs[0]])
        agg_sg = jnp.zeros((N, H), f32).at[ei_sg[1]].add(ps[:, 2 * H:][ei_sg[0]])
        gcn_s = dinv_ss * (agg_ss + us) + bgs
        gcn_g = dinv_gg * (agg_gg + ug) + bgg
        sage_s = agg_gs / cnt_gs + blgs + ps[:, H:2 * H]
        sage_g = agg_sg / cnt_sg + blsg + pg[:, H:2 * H]
        xs = 0.5 * (gcn_s + sage_s)
        xg = 0.5 * (gcn_g + sage_g)

    bs = batch_state.astype(jnp.int32)
    bg = batch_goal.astype(jnp.int32)
    s_sum = jnp.zeros((B, H), f32).at[bs].add(xs)
    g_sum = jnp.zeros((B, H), f32).at[bg].add(xg)
    s_cnt = jnp.maximum(jnp.zeros((B,), f32).at[bs].add(1.0), 1.0)[:, None]
    g_cnt = jnp.maximum(jnp.zeros((B,), f32).at[bg].add(1.0), 1.0)[:, None]
    z = jnp.concatenate([s_sum / s_cnt, g_sum / g_cnt, depth[:, None]], axis=-1)
    zp = jnp.pad(z, ((0, 0), (0, 256 - z.shape[1])))
    wp = jnp.pad(W_out, ((0, 256 - W_out.shape[0]), (0, 127)))
    out = pl.pallas_call(
        _proj_body,
        out_shape=jax.ShapeDtypeStruct((B, 128), f32),
    )(zp, wp, jnp.pad(b_out, (0, 127)).reshape(1, 128))
    return out[:, 0]


# speedup vs baseline: 10.7670x; 4.4621x over previous
import functools

import jax
import jax.numpy as jnp
from jax import lax
from jax.experimental import pallas as pl
from jax.experimental.pallas import tpu as pltpu
from jax.experimental.pallas import tpu_sc as plsc

N = 10000
NPAD = 10240
E = 320000
CHUNK = 128
TCHUNKS = 157
EPAD = TCHUNKS * CHUNK * 16
H = 64
B = 256
NTILE = 16
ZROWS = NPAD // NTILE
BPADROWS = 5
BACC = 272

_mesh = plsc.VectorSubcoreMesh(core_axis_name="c", subcore_axis_name="s")
f32 = jnp.float32
i32 = jnp.int32


def _hist_body(dss, dgs, dgg, dsg, bs2, bg2, zeros_h, ones_h,
               h_ss, h_gs, h_gg, h_sg, bc_s, bc_g,
               idxd, ones_v, zbuf, acc_a, acc_b, bacc):
    c = lax.axis_index("c")
    s = lax.axis_index("s")
    pltpu.sync_copy(zeros_h, zbuf)
    pltpu.sync_copy(ones_h, ones_v)
    pltpu.sync_copy(zbuf, acc_a.at[pl.ds(s * ZROWS, ZROWS)])
    pltpu.sync_copy(zbuf, acc_b.at[pl.ds(s * ZROWS, ZROWS)])

    @pl.when(s == 0)
    def _():
        pltpu.sync_copy(zbuf.at[pl.ds(0, BACC)], bacc)

    plsc.subcore_barrier()

    def run(dst1d, acc, nrows):
        def chunk(j, carry):
            pltpu.sync_copy(dst1d.at[pl.ds((s * nrows + j) * CHUNK, CHUNK)], idxd.at[0])
            pltpu.sync_copy(ones_v, acc.at[idxd.at[0]], add=True)
            return carry
        lax.fori_loop(0, nrows, chunk, 0)

    @pl.when(c == 0)
    def _():
        run(dss, acc_a, TCHUNKS)
        run(dgs, acc_b, TCHUNKS)
        run(bs2, bacc, BPADROWS)

    @pl.when(c == 1)
    def _():
        run(dgg, acc_a, TCHUNKS)
        run(dsg, acc_b, TCHUNKS)
        run(bg2, bacc, BPADROWS)

    plsc.subcore_barrier()
    sl = pl.ds(s * ZROWS, ZROWS)

    @pl.when(c == 0)
    def _():
        pltpu.sync_copy(acc_a.at[sl], h_ss.at[sl])
        pltpu.sync_copy(acc_b.at[sl], h_gs.at[sl])

        @pl.when(s == 0)
        def _():
            pltpu.sync_copy(bacc.at[pl.ds(0, B)], bc_s)

    @pl.when(c == 1)
    def _():
        pltpu.sync_copy(acc_a.at[sl], h_gg.at[sl])
        pltpu.sync_copy(acc_b.at[sl], h_sg.at[sl])

        @pl.when(s == 0)
        def _():
            pltpu.sync_copy(bacc.at[pl.ds(0, B)], bc_g)


_hist_call = functools.partial(
    pl.kernel, _hist_body, mesh=_mesh,
    compiler_params=pltpu.CompilerParams(use_tc_tiling_on_sc=False),
    out_type=[jax.ShapeDtypeStruct((NPAD, 16), f32)] * 4 + [jax.ShapeDtypeStruct((B, 16), f32)] * 2,
    scratch_types=[
        pltpu.VMEM((1, CHUNK), i32),
        pltpu.VMEM((CHUNK, 16), f32),
        pltpu.VMEM((ZROWS, 16), f32),
        pltpu.VMEM_SHARED((NPAD, 16), f32),
        pltpu.VMEM_SHARED((NPAD, 16), f32),
        pltpu.VMEM_SHARED((BACC, 16), f32),
    ],
)


def _agg_body(tab_ss, sss, dss, tab_gs, sgs, dgs, tab_gg, sgg, dgg, tab_sg, ssg, dsg, zeros_h,
              o_ss, o_gs, o_gg, o_sg,
              idxs, idxd, rows, zbuf, acc, sem):
    c = lax.axis_index("c")
    s = lax.axis_index("s")
    sl = pl.ds(s * ZROWS, ZROWS)
    pltpu.sync_copy(zeros_h, zbuf)

    def run(tab, src1d, dst1d, out):
        pltpu.sync_copy(zbuf, acc.at[sl])
        plsc.subcore_barrier()
        base = s * TCHUNKS

        def chunk(j, carry):
            off = (base + j) * CHUNK
            pltpu.sync_copy(src1d.at[pl.ds(off, CHUNK)], idxs.at[0])
            pltpu.sync_copy(dst1d.at[pl.ds(off, CHUNK)], idxd.at[0])
            pltpu.async_copy(tab.at[idxs.at[0]], rows.at[0], sem).wait()
            pltpu.sync_copy(rows.at[0], acc.at[idxd.at[0]], add=True)
            return carry
        lax.fori_loop(0, TCHUNKS, chunk, 0)
        plsc.subcore_barrier()
        pltpu.sync_copy(acc.at[sl], out.at[sl])

    @pl.when(c == 0)
    def _():
        run(tab_ss, sss, dss, o_ss)
        run(tab_gs, sgs, dgs, o_gs)

    @pl.when(c == 1)
    def _():
        run(tab_gg, sgg, dgg, o_gg)
        run(tab_sg, ssg, dsg, o_sg)


_agg_call = functools.partial(
    pl.kernel, _agg_body, mesh=_mesh,
    compiler_params=pltpu.CompilerParams(use_tc_tiling_on_sc=False),
    out_type=[jax.ShapeDtypeStruct((NPAD, H), f32)] * 4,
    scratch_types=[
        pltpu.VMEM((1, CHUNK), i32),
        pltpu.VMEM((1, CHUNK), i32),
        pltpu.VMEM((1, CHUNK, H), f32),
        pltpu.VMEM((ZROWS, H), f32),
        pltpu.VMEM_SHARED((NPAD, H), f32),
        pltpu.SemaphoreType.DMA,
    ],
)


_R = 1000


def _proj0_body(xs_ref, xg_ref, hss_ref, hgg_ref, wcs_ref, wcg_ref,
                us_ref, sr_ref, sl_ref, ug_ref, gr_ref, gl_ref):
    ps = jnp.dot(xs_ref[...], wcs_ref[...], preferred_element_type=f32)
    pg = jnp.dot(xg_ref[...], wcg_ref[...], preferred_element_type=f32)
    dinv_s = lax.rsqrt(hss_ref[...] + 1.0)
    dinv_g = lax.rsqrt(hgg_ref[...] + 1.0)
    us_ref[...] = dinv_s * ps[:, :H]
    sr_ref[...] = ps[:, H:2 * H]
    sl_ref[...] = ps[:, 2 * H:]
    ug_ref[...] = dinv_g * pg[:, :H]
    gr_ref[...] = pg[:, H:2 * H]
    gl_ref[...] = pg[:, 2 * H:]


def _comb1_body(ass_ref, ags_ref, us0_ref, sr0_ref, hss_ref, hgs_ref, bgs_ref, bls_ref,
                agg_ref, asg_ref, ug0_ref, gr0_ref, hgg_ref, hsg_ref, bgg_ref, blg_ref,
                wcs_ref, wcg_ref,
                us_ref, sr_ref, sl_ref, ug_ref, gr_ref, gl_ref):
    dinv_s = lax.rsqrt(hss_ref[...] + 1.0)
    dinv_g = lax.rsqrt(hgg_ref[...] + 1.0)
    ns = 0.5 * (dinv_s * (ass_ref[...] + us0_ref[...]) + bgs_ref[...]
                + ags_ref[...] / jnp.maximum(hgs_ref[...], 1.0) + bls_ref[...] + sr0_ref[...])
    ng = 0.5 * (dinv_g * (agg_ref[...] + ug0_ref[...]) + bgg_ref[...]
                + asg_ref[...] / jnp.maximum(hsg_ref[...], 1.0) + blg_ref[...] + gr0_ref[...])
    ps = jnp.dot(ns, wcs_ref[...], preferred_element_type=f32)
    pg = jnp.dot(ng, wcg_ref[...], preferred_element_type=f32)
    us_ref[...] = dinv_s * ps[:, :H]
    sr_ref[...] = ps[:, H:2 * H]
    sl_ref[...] = ps[:, 2 * H:]
    ug_ref[...] = dinv_g * pg[:, :H]
    gr_ref[...] = pg[:, H:2 * H]
    gl_ref[...] = pg[:, 2 * H:]


def _final_body(ass_ref, ags_ref, us1_ref, sr1_ref, hss_ref, hgs_ref, bgs_ref, bls_ref,
                agg_ref, asg_ref, ug1_ref, gr1_ref, hgg_ref, hsg_ref, bgg_ref, blg_ref,
                bs_ref, bg_ref, bcs_ref, bcg_ref, dep_ref, wos_ref, wog_ref, wod_ref, bo_ref,
                ssum_ref, gsum_ref, out_ref):
    i = pl.program_id(0)
    dinv_s = lax.rsqrt(hss_ref[...] + 1.0)
    dinv_g = lax.rsqrt(hgg_ref[...] + 1.0)
    xs = 0.5 * (dinv_s * (ass_ref[...] + us1_ref[...]) + bgs_ref[...]
                + ags_ref[...] / jnp.maximum(hgs_ref[...], 1.0) + bls_ref[...] + sr1_ref[...])
    xg = 0.5 * (dinv_g * (agg_ref[...] + ug1_ref[...]) + bgg_ref[...]
                + asg_ref[...] / jnp.maximum(hsg_ref[...], 1.0) + blg_ref[...] + gr1_ref[...])
    cols = lax.broadcasted_iota(i32, (_R, B), 1)
    oh_s = (bs_ref[0, 0][:, None] == cols).astype(f32)
    oh_g = (bg_ref[0, 0][:, None] == cols).astype(f32)
    cs = lax.dot_general(oh_s, xs, (((0,), (0,)), ((), ())), preferred_element_type=f32)
    cg = lax.dot_general(oh_g, xg, (((0,), (0,)), ((), ())), preferred_element_type=f32)

    @pl.when(i == 0)
    def _():
        ssum_ref[...] = cs
        gsum_ref[...] = cg

    @pl.when(i > 0)
    def _():
        ssum_ref[...] += cs
        gsum_ref[...] += cg

    @pl.when(i == (N // _R) - 1)
    def _():
        sm = ssum_ref[...] / jnp.maximum(bcs_ref[...], 1.0)
        gm = gsum_ref[...] / jnp.maximum(bcg_ref[...], 1.0)
        out_ref[...] = (jnp.dot(sm, wos_ref[...], preferred_element_type=f32)
                        + jnp.dot(gm, wog_ref[...], preferred_element_type=f32)
                        + dep_ref[...] * wod_ref[...] + bo_ref[...])


def _rowspec(w):
    return pl.BlockSpec((_R, w), lambda i: (i, 0))


def _fullspec(shape):
    return pl.BlockSpec(shape, lambda i: tuple(0 for _ in shape))


def _pad_edges(ei):
    src = jnp.concatenate([ei[0], jnp.zeros((EPAD - E,), i32)])
    dst = jnp.concatenate([ei[1], jnp.full((EPAD - E,), N, i32)])
    return src, dst


def kernel(x_state, x_goal, ei_ss, ei_gg, ei_sg, ei_gs, batch_state, batch_goal, depth,
           W_gcn_s_0, b_gcn_s_0, W_gcn_g_0, b_gcn_g_0, Wl_sg_0, bl_sg_0, Wr_sg_0,
           Wl_gs_0, bl_gs_0, Wr_gs_0,
           W_gcn_s_1, b_gcn_s_1, W_gcn_g_1, b_gcn_g_1, Wl_sg_1, bl_sg_1, Wr_sg_1,
           Wl_gs_1, bl_gs_1, Wr_gs_1, W_out, b_out):
    sss, dss = _pad_edges(ei_ss.astype(i32))
    sgg, dgg = _pad_edges(ei_gg.astype(i32))
    ssg, dsg = _pad_edges(ei_sg.astype(i32))
    sgs, dgs = _pad_edges(ei_gs.astype(i32))
    NB2 = BPADROWS * NTILE
    bs2 = jnp.concatenate([batch_state.astype(i32), jnp.full((NB2 * CHUNK - N,), B, i32)])
    bg2 = jnp.concatenate([batch_goal.astype(i32), jnp.full((NB2 * CHUNK - N,), B, i32)])

    z16 = jnp.zeros((ZROWS, 16), f32)
    o16 = jnp.ones((CHUNK, 16), f32)
    h_ss, h_gs, h_gg, h_sg, bc_s, bc_g = _hist_call()(dss, dgs, dgg, dsg, bs2, bg2, z16, o16)
    hss = h_ss[:, :1]
    hgs = h_gs[:, :1]
    hgg = h_gg[:, :1]
    hsg = h_sg[:, :1]

    wcs0 = jnp.concatenate([W_gcn_s_0, Wr_gs_0, Wl_sg_0], axis=1)
    wcg0 = jnp.concatenate([W_gcn_g_0, Wr_sg_0, Wl_gs_0], axis=1)
    grid = N // _R
    outH = [jax.ShapeDtypeStruct((N, H), f32)] * 6
    us0, sr0, sl0, ug0, gr0, gl0 = pl.pallas_call(
        _proj0_body,
        grid=(grid,),
        in_specs=[_rowspec(128), _rowspec(128), _rowspec(1), _rowspec(1),
                  _fullspec((128, 3 * H)), _fullspec((128, 3 * H))],
        out_specs=[_rowspec(H)] * 6,
        out_shape=outH,
    )(x_state, x_goal, hss, hgg, wcs0, wcg0)

    zH = jnp.zeros((ZROWS, H), f32)
    a_ss0, a_gs0, a_gg0, a_sg0 = _agg_call()(
        us0, sss, dss, gl0, sgs, dgs, ug0, sgg, dgg, sl0, ssg, dsg, zH)

    wcs1 = jnp.concatenate([W_gcn_s_1, Wr_gs_1, Wl_sg_1], axis=1)
    wcg1 = jnp.concatenate([W_gcn_g_1, Wr_sg_1, Wl_gs_1], axis=1)
    bgs0 = b_gcn_s_0.reshape(1, H)
    bls0 = bl_gs_0.reshape(1, H)
    bgg0 = b_gcn_g_0.reshape(1, H)
    blg0 = bl_sg_0.reshape(1, H)
    us1, sr1, sl1, ug1, gr1, gl1 = pl.pallas_call(
        _comb1_body,
        grid=(grid,),
        in_specs=[_rowspec(H), _rowspec(H), _rowspec(H), _rowspec(H), _rowspec(1), _rowspec(1),
                  _fullspec((1, H)), _fullspec((1, H)),
                  _rowspec(H), _rowspec(H), _rowspec(H), _rowspec(H), _rowspec(1), _rowspec(1),
                  _fullspec((1, H)), _fullspec((1, H)),
                  _fullspec((H, 3 * H)), _fullspec((H, 3 * H))],
        out_specs=[_rowspec(H)] * 6,
        out_shape=outH,
    )(a_ss0, a_gs0, us0, sr0, hss, hgs, bgs0, bls0,
      a_gg0, a_sg0, ug0, gr0, hgg, hsg, bgg0, blg0, wcs1, wcg1)

    a_ss1, a_gs1, a_gg1, a_sg1 = _agg_call()(
        us1, sss, dss, gl1, sgs, dgs, ug1, sgg, dgg, sl1, ssg, dsg, zH)

    bs3 = batch_state.astype(i32).reshape(grid, 1, _R)
    bg3 = batch_goal.astype(i32).reshape(grid, 1, _R)
    bgs1 = b_gcn_s_1.reshape(1, H)
    bls1 = bl_gs_1.reshape(1, H)
    bgg1 = b_gcn_g_1.reshape(1, H)
    blg1 = bl_sg_1.reshape(1, H)
    _, _, out = pl.pallas_call(
        _final_body,
        grid=(grid,),
        in_specs=[_rowspec(H), _rowspec(H), _rowspec(H), _rowspec(H), _rowspec(1), _rowspec(1),
                  _fullspec((1, H)), _fullspec((1, H)),
                  _rowspec(H), _rowspec(H), _rowspec(H), _rowspec(H), _rowspec(1), _rowspec(1),
                  _fullspec((1, H)), _fullspec((1, H)),
                  pl.BlockSpec((1, 1, _R), lambda i: (i, 0, 0)),
                  pl.BlockSpec((1, 1, _R), lambda i: (i, 0, 0)),
                  _fullspec((B, 1)), _fullspec((B, 1)), _fullspec((B, 1)),
                  _fullspec((H, 1)), _fullspec((H, 1)), _fullspec((1, 1)), _fullspec((1, 1))],
        out_specs=[_fullspec((B, H)), _fullspec((B, H)), _fullspec((B, 1))],
        out_shape=[jax.ShapeDtypeStruct((B, H), f32), jax.ShapeDtypeStruct((B, H), f32),
                   jax.ShapeDtypeStruct((B, 1), f32)],
    )(a_ss1, a_gs1, us1, sr1, hss, hgs, bgs1, bls1,
      a_gg1, a_sg1, ug1, gr1, hgg, hsg, bgg1, blg1,
      bs3, bg3, bc_s[:, :1], bc_g[:, :1], depth.reshape(B, 1),
      W_out[:H], W_out[H:2 * H], W_out[2 * H:].reshape(1, 1), b_out.reshape(1, 1))
    return out[:, 0]


# speedup vs baseline: 12.7861x; 1.1875x over previous
import functools

import jax
import jax.numpy as jnp
from jax import lax
from jax.experimental import pallas as pl
from jax.experimental.pallas import tpu as pltpu
from jax.experimental.pallas import tpu_sc as plsc

N = 10000
NPAD = 10240
E = 320000
CHUNK = 128
K = 4
TCHUNKS = 160
GROUPS = TCHUNKS // K
EPAD = TCHUNKS * CHUNK * 16
H = 64
B = 256
NTILE = 16
ZROWS = NPAD // NTILE
ZB = 80
BPADROWS = 5
BACC = 272

_mesh = plsc.VectorSubcoreMesh(core_axis_name="c", subcore_axis_name="s")
f32 = jnp.float32
i32 = jnp.int32


def _hist_body(dss, dgs, dgg, dsg, bs2, bg2, zeros_h, ones_h,
               h_ss, h_gs, h_gg, h_sg, bc_s, bc_g,
               idxd, ones_v, zbuf, acc_a, acc_b, bacc, hs0, hs1):
    c = lax.axis_index("c")
    s = lax.axis_index("s")
    pltpu.sync_copy(zeros_h, zbuf)
    pltpu.sync_copy(ones_h, ones_v)
    pltpu.sync_copy(zbuf, acc_a.at[pl.ds(s * ZROWS, ZROWS)])
    pltpu.sync_copy(zbuf, acc_b.at[pl.ds(s * ZROWS, ZROWS)])

    @pl.when(s == 0)
    def _():
        pltpu.sync_copy(zbuf.at[pl.ds(0, BACC)], bacc)

    plsc.subcore_barrier()

    def run(dst2d, acc):
        base = s * GROUPS

        def stage(g, b):
            pltpu.sync_copy(dst2d.at[pl.ds((base + g) * K, K)], idxd.at[b])

        def fire(b, sem):
            for k in range(K):
                pltpu.async_copy(ones_v, acc.at[idxd.at[b, k]], sem, add=True)

        def drain(b, sem):
            for k in range(K):
                pltpu.make_async_copy(ones_v, acc.at[idxd.at[b, k]], sem).wait()

        stage(0, 0)

        def body(r, carry):
            for b, sem, nsem in ((0, hs0, hs1), (1, hs1, hs0)):
                g = 2 * r + b
                nb = 1 - b
                fire(b, sem)

                @pl.when(g >= 1)
                def _():
                    drain(nb, nsem)

                @pl.when(g + 1 < GROUPS)
                def _():
                    stage(g + 1, nb)
            return carry
        lax.fori_loop(0, GROUPS // 2, body, 0)
        drain(1, hs1)

    def runb(bat1d, acc, nrows):
        def chunk(j, carry):
            pltpu.sync_copy(bat1d.at[pl.ds((s * nrows + j) * CHUNK, CHUNK)], idxd.at[0, 0])
            pltpu.sync_copy(ones_v, acc.at[idxd.at[0, 0]], add=True)
            return carry
        lax.fori_loop(0, nrows, chunk, 0)

    @pl.when(c == 0)
    def _():
        run(dss, acc_a)
        run(dgs, acc_b)
        runb(bs2, bacc, BPADROWS)

    @pl.when(c == 1)
    def _():
        run(dgg, acc_a)
        run(dsg, acc_b)
        runb(bg2, bacc, BPADROWS)

    plsc.subcore_barrier()
    sl = pl.ds(s * ZROWS, ZROWS)

    @pl.when(c == 0)
    def _():
        pltpu.sync_copy(acc_a.at[sl], h_ss.at[sl])
        pltpu.sync_copy(acc_b.at[sl], h_gs.at[sl])

        @pl.when(s == 0)
        def _():
            pltpu.sync_copy(bacc.at[pl.ds(0, B)], bc_s)

    @pl.when(c == 1)
    def _():
        pltpu.sync_copy(acc_a.at[sl], h_gg.at[sl])
        pltpu.sync_copy(acc_b.at[sl], h_sg.at[sl])

        @pl.when(s == 0)
        def _():
            pltpu.sync_copy(bacc.at[pl.ds(0, B)], bc_g)


_hist_call = functools.partial(
    pl.kernel, _hist_body, mesh=_mesh,
    compiler_params=pltpu.CompilerParams(use_tc_tiling_on_sc=False),
    out_type=[jax.ShapeDtypeStruct((NPAD, 16), f32)] * 4 + [jax.ShapeDtypeStruct((B, 16), f32)] * 2,
    scratch_types=[
        pltpu.VMEM((2, K, CHUNK), i32),
        pltpu.VMEM((CHUNK, 16), f32),
        pltpu.VMEM((ZROWS, 16), f32),
        pltpu.VMEM_SHARED((NPAD, 16), f32),
        pltpu.VMEM_SHARED((NPAD, 16), f32),
        pltpu.VMEM_SHARED((BACC, 16), f32),
        pltpu.SemaphoreType.DMA,
        pltpu.SemaphoreType.DMA,
    ],
)


def _agg_body(tab_ss, css, tab_gs, cgs, tab_gg, cgg, tab_sg, csg, zeros_h,
              o_ss, o_gs, o_gg, o_sg,
              idxv, rows, zbuf, acc, gs0, gs1, ss0, ss1):
    c = lax.axis_index("c")
    s = lax.axis_index("s")
    sl = pl.ds(s * ZROWS, ZROWS)
    pltpu.sync_copy(zeros_h, zbuf)

    def run(tab, comb2d, out):
        for z in range(ZROWS // ZB):
            pltpu.sync_copy(zbuf, acc.at[pl.ds(s * ZROWS + z * ZB, ZB)])
        plsc.subcore_barrier()
        base = s * GROUPS

        def stage(g, b):
            pltpu.sync_copy(comb2d.at[pl.ds((base + g) * 2 * K, 2 * K)], idxv.at[b])

        def gfire(b, sem):
            for k in range(K):
                pltpu.async_copy(tab.at[idxv.at[b, k]], rows.at[b, k], sem)

        def gdrain(b, sem):
            for k in range(K):
                pltpu.make_async_copy(tab.at[idxv.at[b, k]], rows.at[b, k], sem).wait()

        def sfire(b, sem):
            for k in range(K):
                pltpu.async_copy(rows.at[b, k], acc.at[idxv.at[b, K + k]], sem, add=True)

        def sdrain(b, sem):
            for k in range(K):
                pltpu.make_async_copy(rows.at[b, k], acc.at[idxv.at[b, K + k]], sem).wait()

        stage(0, 0)
        gfire(0, gs0)

        def body(r, carry):
            for b, gsem, ngsem, ssem, nssem in ((0, gs0, gs1, ss0, ss1),
                                                (1, gs1, gs0, ss1, ss0)):
                g = 2 * r + b
                nb = 1 - b
                gdrain(b, gsem)
                sfire(b, ssem)

                @pl.when(g >= 1)
                def _():
                    sdrain(nb, nssem)

                @pl.when(g + 1 < GROUPS)
                def _():
                    stage(g + 1, nb)
                    gfire(nb, ngsem)
            return carry
        lax.fori_loop(0, GROUPS // 2, body, 0)
        sdrain(1, ss1)
        plsc.subcore_barrier()
        pltpu.sync_copy(acc.at[sl], out.at[sl])

    @pl.when(c == 0)
    def _():
        run(tab_ss, css, o_ss)
        run(tab_gs, cgs, o_gs)

    @pl.when(c == 1)
    def _():
        run(tab_gg, cgg, o_gg)
        run(tab_sg, csg, o_sg)


_agg_call = functools.partial(
    pl.kernel, _agg_body, mesh=_mesh,
    compiler_params=pltpu.CompilerParams(use_tc_tiling_on_sc=False),
    out_type=[jax.ShapeDtypeStruct((NPAD, H), f32)] * 4,
    scratch_types=[
        pltpu.VMEM((2, 2 * K, CHUNK), i32),
        pltpu.VMEM((2, K, CHUNK, H), f32),
        pltpu.VMEM((ZB, H), f32),
        pltpu.VMEM_SHARED((NPAD, H), f32),
        pltpu.SemaphoreType.DMA,
        pltpu.SemaphoreType.DMA,
        pltpu.SemaphoreType.DMA,
        pltpu.SemaphoreType.DMA,
    ],
)


_R = 1000


def _proj0_body(xs_ref, xg_ref, hss_ref, hgg_ref, wcs_ref, wcg_ref,
                us_ref, sr_ref, sl_ref, ug_ref, gr_ref, gl_ref):
    ps = jnp.dot(xs_ref[...], wcs_ref[...], preferred_element_type=f32)
    pg = jnp.dot(xg_ref[...], wcg_ref[...], preferred_element_type=f32)
    dinv_s = lax.rsqrt(hss_ref[...] + 1.0)
    dinv_g = lax.rsqrt(hgg_ref[...] + 1.0)
    us_ref[...] = dinv_s * ps[:, :H]
    sr_ref[...] = ps[:, H:2 * H]
    sl_ref[...] = ps[:, 2 * H:]
    ug_ref[...] = dinv_g * pg[:, :H]
    gr_ref[...] = pg[:, H:2 * H]
    gl_ref[...] = pg[:, 2 * H:]


def _comb1_body(ass_ref, ags_ref, us0_ref, sr0_ref, hss_ref, hgs_ref, bgs_ref, bls_ref,
                agg_ref, asg_ref, ug0_ref, gr0_ref, hgg_ref, hsg_ref, bgg_ref, blg_ref,
                wcs_ref, wcg_ref,
                us_ref, sr_ref, sl_ref, ug_ref, gr_ref, gl_ref):
    dinv_s = lax.rsqrt(hss_ref[...] + 1.0)
    dinv_g = lax.rsqrt(hgg_ref[...] + 1.0)
    ns = 0.5 * (dinv_s * (ass_ref[...] + us0_ref[...]) + bgs_ref[...]
                + ags_ref[...] / jnp.maximum(hgs_ref[...], 1.0) + bls_ref[...] + sr0_ref[...])
    ng = 0.5 * (dinv_g * (agg_ref[...] + ug0_ref[...]) + bgg_ref[...]
                + asg_ref[...] / jnp.maximum(hsg_ref[...], 1.0) + blg_ref[...] + gr0_ref[...])
    ps = jnp.dot(ns, wcs_ref[...], preferred_element_type=f32)
    pg = jnp.dot(ng, wcg_ref[...], preferred_element_type=f32)
    us_ref[...] = dinv_s * ps[:, :H]
    sr_ref[...] = ps[:, H:2 * H]
    sl_ref[...] = ps[:, 2 * H:]
    ug_ref[...] = dinv_g * pg[:, :H]
    gr_ref[...] = pg[:, H:2 * H]
    gl_ref[...] = pg[:, 2 * H:]


def _final_body(ass_ref, ags_ref, us1_ref, sr1_ref, hss_ref, hgs_ref, bgs_ref, bls_ref,
                agg_ref, asg_ref, ug1_ref, gr1_ref, hgg_ref, hsg_ref, bgg_ref, blg_ref,
                bs_ref, bg_ref, bcs_ref, bcg_ref, dep_ref, wos_ref, wog_ref, wod_ref, bo_ref,
                ssum_ref, gsum_ref, out_ref):
    i = pl.program_id(0)
    dinv_s = lax.rsqrt(hss_ref[...] + 1.0)
    dinv_g = lax.rsqrt(hgg_ref[...] + 1.0)
    xs = 0.5 * (dinv_s * (ass_ref[...] + us1_ref[...]) + bgs_ref[...]
                + ags_ref[...] / jnp.maximum(hgs_ref[...], 1.0) + bls_ref[...] + sr1_ref[...])
    xg = 0.5 * (dinv_g * (agg_ref[...] + ug1_ref[...]) + bgg_ref[...]
                + asg_ref[...] / jnp.maximum(hsg_ref[...], 1.0) + blg_ref[...] + gr1_ref[...])
    cols = lax.broadcasted_iota(i32, (_R, B), 1)
    oh_s = (bs_ref[0, 0][:, None] == cols).astype(f32)
    oh_g = (bg_ref[0, 0][:, None] == cols).astype(f32)
    cs = lax.dot_general(oh_s, xs, (((0,), (0,)), ((), ())), preferred_element_type=f32)
    cg = lax.dot_general(oh_g, xg, (((0,), (0,)), ((), ())), preferred_element_type=f32)

    @pl.when(i == 0)
    def _():
        ssum_ref[...] = cs
        gsum_ref[...] = cg

    @pl.when(i > 0)
    def _():
        ssum_ref[...] += cs
        gsum_ref[...] += cg

    @pl.when(i == (N // _R) - 1)
    def _():
        sm = ssum_ref[...] / jnp.maximum(bcs_ref[...], 1.0)
        gm = gsum_ref[...] / jnp.maximum(bcg_ref[...], 1.0)
        out_ref[...] = (jnp.dot(sm, wos_ref[...], preferred_element_type=f32)
                        + jnp.dot(gm, wog_ref[...], preferred_element_type=f32)
                        + dep_ref[...] * wod_ref[...] + bo_ref[...])


def _rowspec(w):
    return pl.BlockSpec((_R, w), lambda i: (i, 0))


def _fullspec(shape):
    return pl.BlockSpec(shape, lambda i: tuple(0 for _ in shape))


def _pad_edges(ei):
    src = jnp.concatenate([ei[0], jnp.zeros((EPAD - E,), i32)])
    dst = jnp.concatenate([ei[1], jnp.full((EPAD - E,), N, i32)])
    comb = jnp.stack([src.reshape(-1, K * CHUNK), dst.reshape(-1, K * CHUNK)],
                     axis=1).reshape(-1, CHUNK)
    return comb, dst.reshape(-1, CHUNK)


def kernel(x_state, x_goal, ei_ss, ei_gg, ei_sg, ei_gs, batch_state, batch_goal, depth,
           W_gcn_s_0, b_gcn_s_0, W_gcn_g_0, b_gcn_g_0, Wl_sg_0, bl_sg_0, Wr_sg_0,
           Wl_gs_0, bl_gs_0, Wr_gs_0,
           W_gcn_s_1, b_gcn_s_1, W_gcn_g_1, b_gcn_g_1, Wl_sg_1, bl_sg_1, Wr_sg_1,
           Wl_gs_1, bl_gs_1, Wr_gs_1, W_out, b_out):
    css, dss = _pad_edges(ei_ss.astype(i32))
    cgg, dgg = _pad_edges(ei_gg.astype(i32))
    csg, dsg = _pad_edges(ei_sg.astype(i32))
    cgs, dgs = _pad_edges(ei_gs.astype(i32))
    NB2 = BPADROWS * NTILE
    bs2 = jnp.concatenate([batch_state.astype(i32), jnp.full((NB2 * CHUNK - N,), B, i32)])
    bg2 = jnp.concatenate([batch_goal.astype(i32), jnp.full((NB2 * CHUNK - N,), B, i32)])

    z16 = jnp.zeros((ZROWS, 16), f32)
    o16 = jnp.ones((CHUNK, 16), f32)
    h_ss, h_gs, h_gg, h_sg, bc_s, bc_g = _hist_call()(dss, dgs, dgg, dsg, bs2, bg2, z16, o16)
    hss = h_ss[:, :1]
    hgs = h_gs[:, :1]
    hgg = h_gg[:, :1]
    hsg = h_sg[:, :1]

    wcs0 = jnp.concatenate([W_gcn_s_0, Wr_gs_0, Wl_sg_0], axis=1)
    wcg0 = jnp.concatenate([W_gcn_g_0, Wr_sg_0, Wl_gs_0], axis=1)
    grid = N // _R
    outH = [jax.ShapeDtypeStruct((N, H), f32)] * 6
    us0, sr0, sl0, ug0, gr0, gl0 = pl.pallas_call(
        _proj0_body,
        grid=(grid,),
        in_specs=[_rowspec(128), _rowspec(128), _rowspec(1), _rowspec(1),
                  _fullspec((128, 3 * H)), _fullspec((128, 3 * H))],
        out_specs=[_rowspec(H)] * 6,
        out_shape=outH,
    )(x_state, x_goal, hss, hgg, wcs0, wcg0)

    zH = jnp.zeros((ZB, H), f32)
    a_ss0, a_gs0, a_gg0, a_sg0 = _agg_call()(
        us0, css, gl0, cgs, ug0, cgg, sl0, csg, zH)

    wcs1 = jnp.concatenate([W_gcn_s_1, Wr_gs_1, Wl_sg_1], axis=1)
    wcg1 = jnp.concatenate([W_gcn_g_1, Wr_sg_1, Wl_gs_1], axis=1)
    bgs0 = b_gcn_s_0.reshape(1, H)
    bls0 = bl_gs_0.reshape(1, H)
    bgg0 = b_gcn_g_0.reshape(1, H)
    blg0 = bl_sg_0.reshape(1, H)
    us1, sr1, sl1, ug1, gr1, gl1 = pl.pallas_call(
        _comb1_body,
        grid=(grid,),
        in_specs=[_rowspec(H), _rowspec(H), _rowspec(H), _rowspec(H), _rowspec(1), _rowspec(1),
                  _fullspec((1, H)), _fullspec((1, H)),
                  _rowspec(H), _rowspec(H), _rowspec(H), _rowspec(H), _rowspec(1), _rowspec(1),
                  _fullspec((1, H)), _fullspec((1, H)),
                  _fullspec((H, 3 * H)), _fullspec((H, 3 * H))],
        out_specs=[_rowspec(H)] * 6,
        out_shape=outH,
    )(a_ss0, a_gs0, us0, sr0, hss, hgs, bgs0, bls0,
      a_gg0, a_sg0, ug0, gr0, hgg, hsg, bgg0, blg0, wcs1, wcg1)

    a_ss1, a_gs1, a_gg1, a_sg1 = _agg_call()(
        us1, css, gl1, cgs, ug1, cgg, sl1, csg, zH)

    bs3 = batch_state.astype(i32).reshape(grid, 1, _R)
    bg3 = batch_goal.astype(i32).reshape(grid, 1, _R)
    bgs1 = b_gcn_s_1.reshape(1, H)
    bls1 = bl_gs_1.reshape(1, H)
    bgg1 = b_gcn_g_1.reshape(1, H)
    blg1 = bl_sg_1.reshape(1, H)
    _, _, out = pl.pallas_call(
        _final_body,
        grid=(grid,),
        in_specs=[_rowspec(H), _rowspec(H), _rowspec(H), _rowspec(H), _rowspec(1), _rowspec(1),
                  _fullspec((1, H)), _fullspec((1, H)),
                  _rowspec(H), _rowspec(H), _rowspec(H), _rowspec(H), _rowspec(1), _rowspec(1),
                  _fullspec((1, H)), _fullspec((1, H)),
                  pl.BlockSpec((1, 1, _R), lambda i: (i, 0, 0)),
                  pl.BlockSpec((1, 1, _R), lambda i: (i, 0, 0)),
                  _fullspec((B, 1)), _fullspec((B, 1)), _fullspec((B, 1)),
                  _fullspec((H, 1)), _fullspec((H, 1)), _fullspec((1, 1)), _fullspec((1, 1))],
        out_specs=[_fullspec((B, H)), _fullspec((B, H)), _fullspec((B, 1))],
        out_shape=[jax.ShapeDtypeStruct((B, H), f32), jax.ShapeDtypeStruct((B, H), f32),
                   jax.ShapeDtypeStruct((B, 1), f32)],
    )(a_ss1, a_gs1, us1, sr1, hss, hgs, bgs1, bls1,
      a_gg1, a_sg1, ug1, gr1, hgg, hsg, bgg1, blg1,
      bs3, bg3, bc_s[:, :1], bc_g[:, :1], depth.reshape(B, 1),
      W_out[:H], W_out[H:2 * H], W_out[2 * H:].reshape(1, 1), b_out.reshape(1, 1))
    return out[:, 0]


# speedup vs baseline: 13.2795x; 1.0386x over previous
import functools

import jax
import jax.numpy as jnp
from jax import lax
from jax.experimental import pallas as pl
from jax.experimental.pallas import tpu as pltpu
from jax.experimental.pallas import tpu_sc as plsc

N = 10000
NPAD = 10240
E = 320000
CHUNK = 128
K = 4
TCHUNKS = 160
GROUPS = TCHUNKS // K
EPAD = TCHUNKS * CHUNK * 16
H = 64
B = 256
NTILE = 16
ZROWS = NPAD // NTILE
ZB = 80
BPADROWS = 5
BACC = 272

_mesh = plsc.VectorSubcoreMesh(core_axis_name="c", subcore_axis_name="s")
f32 = jnp.float32
i32 = jnp.int32


def _hist_body(dss, dgs, dgg, dsg, bs2, bg2, zeros_h, ones_h,
               h_ss, h_gs, h_gg, h_sg, bc_s, bc_g,
               idxd, ones_v, zbuf, acc_a, acc_b, bacc, hs0, hs1):
    c = lax.axis_index("c")
    s = lax.axis_index("s")
    pltpu.sync_copy(zeros_h, zbuf)
    pltpu.sync_copy(ones_h, ones_v)
    pltpu.sync_copy(zbuf, acc_a.at[pl.ds(s * ZROWS, ZROWS)])
    pltpu.sync_copy(zbuf, acc_b.at[pl.ds(s * ZROWS, ZROWS)])

    @pl.when(s == 0)
    def _():
        pltpu.sync_copy(zbuf.at[pl.ds(0, BACC)], bacc)

    plsc.subcore_barrier()

    def run(dst2d, acc):
        base = s * GROUPS

        def stage(g, b):
            pltpu.sync_copy(dst2d.at[pl.ds((base + g) * K, K)], idxd.at[b])

        def fire(b, sem):
            for k in range(K):
                pltpu.async_copy(ones_v, acc.at[idxd.at[b, k]], sem, add=True)

        def drain(b, sem):
            for k in range(K):
                pltpu.make_async_copy(ones_v, acc.at[idxd.at[b, k]], sem).wait()

        stage(0, 0)

        def body(r, carry):
            for b, sem, nsem in ((0, hs0, hs1), (1, hs1, hs0)):
                g = 2 * r + b
                nb = 1 - b
                fire(b, sem)

                @pl.when(g >= 1)
                def _():
                    drain(nb, nsem)

                @pl.when(g + 1 < GROUPS)
                def _():
                    stage(g + 1, nb)
            return carry
        lax.fori_loop(0, GROUPS // 2, body, 0)
        drain(1, hs1)

    def runb(bat1d, acc, nrows):
        def chunk(j, carry):
            pltpu.sync_copy(bat1d.at[pl.ds((s * nrows + j) * CHUNK, CHUNK)], idxd.at[0, 0])
            pltpu.sync_copy(ones_v, acc.at[idxd.at[0, 0]], add=True)
            return carry
        lax.fori_loop(0, nrows, chunk, 0)

    @pl.when(c == 0)
    def _():
        run(dss, acc_a)
        run(dgs, acc_b)
        runb(bs2, bacc, BPADROWS)

    @pl.when(c == 1)
    def _():
        run(dgg, acc_a)
        run(dsg, acc_b)
        runb(bg2, bacc, BPADROWS)

    plsc.subcore_barrier()
    sl = pl.ds(s * ZROWS, ZROWS)

    @pl.when(c == 0)
    def _():
        pltpu.sync_copy(acc_a.at[sl], h_ss.at[sl])
        pltpu.sync_copy(acc_b.at[sl], h_gs.at[sl])

        @pl.when(s == 0)
        def _():
            pltpu.sync_copy(bacc.at[pl.ds(0, B)], bc_s)

    @pl.when(c == 1)
    def _():
        pltpu.sync_copy(acc_a.at[sl], h_gg.at[sl])
        pltpu.sync_copy(acc_b.at[sl], h_sg.at[sl])

        @pl.when(s == 0)
        def _():
            pltpu.sync_copy(bacc.at[pl.ds(0, B)], bc_g)


_hist_call = functools.partial(
    pl.kernel, _hist_body, mesh=_mesh,
    compiler_params=pltpu.CompilerParams(use_tc_tiling_on_sc=False),
    out_type=[jax.ShapeDtypeStruct((NPAD, 16), f32)] * 4 + [jax.ShapeDtypeStruct((B, 16), f32)] * 2,
    scratch_types=[
        pltpu.VMEM((2, K, CHUNK), i32),
        pltpu.VMEM((CHUNK, 16), f32),
        pltpu.VMEM((ZROWS, 16), f32),
        pltpu.VMEM_SHARED((NPAD, 16), f32),
        pltpu.VMEM_SHARED((NPAD, 16), f32),
        pltpu.VMEM_SHARED((BACC, 16), f32),
        pltpu.SemaphoreType.DMA,
        pltpu.SemaphoreType.DMA,
    ],
)


def _agg_body(tab_ss, css, tab_gs, cgs, tab_gg, cgg, tab_sg, csg, zeros_h,
              o_ss, o_gs, o_gg, o_sg,
              idxv, rows, acc, is0, is1, is2, is3, gs0, gs1, ss0, ss1):
    c = lax.axis_index("c")
    s = lax.axis_index("s")
    sl = pl.ds(s * ZROWS, ZROWS)
    isems = (is0, is1, is2, is3)
    gsems = (gs0, gs1)
    ssems = (ss0, ss1)

    def run(tab, comb2d, out):
        pltpu.sync_copy(zeros_h.at[sl], acc.at[sl])
        plsc.subcore_barrier()
        base = s * GROUPS

        def istart(g, ib):
            pltpu.async_copy(comb2d.at[pl.ds((base + g) * 2 * K, 2 * K)],
                             idxv.at[ib], isems[ib])

        def iwait(g, ib):
            pltpu.make_async_copy(comb2d.at[pl.ds((base + g) * 2 * K, 2 * K)],
                                  idxv.at[ib], isems[ib]).wait()

        def gfire(ib, b):
            for k in range(K):
                pltpu.async_copy(tab.at[idxv.at[ib, k]], rows.at[b, k], gsems[b])

        def gdrain(ib, b):
            for k in range(K):
                pltpu.make_async_copy(tab.at[idxv.at[ib, k]], rows.at[b, k], gsems[b]).wait()

        def sfire(ib, b):
            for k in range(K):
                pltpu.async_copy(rows.at[b, k], acc.at[idxv.at[ib, K + k]], ssems[b], add=True)

        def sdrain(ib, b):
            for k in range(K):
                pltpu.make_async_copy(rows.at[b, k], acc.at[idxv.at[ib, K + k]], ssems[b]).wait()

        istart(0, 0)
        istart(1, 1)
        iwait(0, 0)
        gfire(0, 0)

        def body(r, carry):
            for u in range(4):
                g = 4 * r + u
                b = u % 2
                nb = 1 - b
                ib = u
                nib = (u + 1) % 4

                @pl.when(g + 2 < GROUPS)
                def _():
                    istart(g + 2, (u + 2) % 4)
                gdrain(ib, b)
                sfire(ib, b)

                @pl.when(g >= 1)
                def _():
                    sdrain((u - 1) % 4, nb)

                @pl.when(g + 1 < GROUPS)
                def _():
                    iwait(g + 1, nib)
                    gfire(nib, nb)
            return carry
        lax.fori_loop(0, GROUPS // 4, body, 0)
        sdrain(3, 1)
        plsc.subcore_barrier()
        pltpu.sync_copy(acc.at[sl], out.at[sl])

    @pl.when(c == 0)
    def _():
        run(tab_ss, css, o_ss)
        run(tab_gs, cgs, o_gs)

    @pl.when(c == 1)
    def _():
        run(tab_gg, cgg, o_gg)
        run(tab_sg, csg, o_sg)


_agg_call = functools.partial(
    pl.kernel, _agg_body, mesh=_mesh,
    compiler_params=pltpu.CompilerParams(use_tc_tiling_on_sc=False),
    out_type=[jax.ShapeDtypeStruct((NPAD, H), f32)] * 4,
    scratch_types=[
        pltpu.VMEM((4, 2 * K, CHUNK), i32),
        pltpu.VMEM((2, K, CHUNK, H), f32),
        pltpu.VMEM_SHARED((NPAD, H), f32),
    ] + [pltpu.SemaphoreType.DMA] * 8,
)


_R = 1000


def _proj0_body(xs_ref, xg_ref, hss_ref, hgg_ref, wcs_ref, wcg_ref,
                us_ref, sr_ref, sl_ref, ug_ref, gr_ref, gl_ref):
    ps = jnp.dot(xs_ref[...], wcs_ref[...], preferred_element_type=f32)
    pg = jnp.dot(xg_ref[...], wcg_ref[...], preferred_element_type=f32)
    dinv_s = lax.rsqrt(hss_ref[...] + 1.0)
    dinv_g = lax.rsqrt(hgg_ref[...] + 1.0)
    us_ref[...] = dinv_s * ps[:, :H]
    sr_ref[...] = ps[:, H:2 * H]
    sl_ref[...] = ps[:, 2 * H:]
    ug_ref[...] = dinv_g * pg[:, :H]
    gr_ref[...] = pg[:, H:2 * H]
    gl_ref[...] = pg[:, 2 * H:]


def _comb1_body(ass_ref, ags_ref, us0_ref, sr0_ref, hss_ref, hgs_ref, bgs_ref, bls_ref,
                agg_ref, asg_ref, ug0_ref, gr0_ref, hgg_ref, hsg_ref, bgg_ref, blg_ref,
                wcs_ref, wcg_ref,
                us_ref, sr_ref, sl_ref, ug_ref, gr_ref, gl_ref):
    dinv_s = lax.rsqrt(hss_ref[...] + 1.0)
    dinv_g = lax.rsqrt(hgg_ref[...] + 1.0)
    ns = 0.5 * (dinv_s * (ass_ref[...] + us0_ref[...]) + bgs_ref[...]
                + ags_ref[...] / jnp.maximum(hgs_ref[...], 1.0) + bls_ref[...] + sr0_ref[...])
    ng = 0.5 * (dinv_g * (agg_ref[...] + ug0_ref[...]) + bgg_ref[...]
                + asg_ref[...] / jnp.maximum(hsg_ref[...], 1.0) + blg_ref[...] + gr0_ref[...])
    ps = jnp.dot(ns, wcs_ref[...], preferred_element_type=f32)
    pg = jnp.dot(ng, wcg_ref[...], preferred_element_type=f32)
    us_ref[...] = dinv_s * ps[:, :H]
    sr_ref[...] = ps[:, H:2 * H]
    sl_ref[...] = ps[:, 2 * H:]
    ug_ref[...] = dinv_g * pg[:, :H]
    gr_ref[...] = pg[:, H:2 * H]
    gl_ref[...] = pg[:, 2 * H:]


def _final_body(ass_ref, ags_ref, us1_ref, sr1_ref, hss_ref, hgs_ref, bgs_ref, bls_ref,
                agg_ref, asg_ref, ug1_ref, gr1_ref, hgg_ref, hsg_ref, bgg_ref, blg_ref,
                bs_ref, bg_ref, bcs_ref, bcg_ref, dep_ref, wos_ref, wog_ref, wod_ref, bo_ref,
                ssum_ref, gsum_ref, out_ref):
    i = pl.program_id(0)
    dinv_s = lax.rsqrt(hss_ref[...] + 1.0)
    dinv_g = lax.rsqrt(hgg_ref[...] + 1.0)
    xs = 0.5 * (dinv_s * (ass_ref[...] + us1_ref[...]) + bgs_ref[...]
                + ags_ref[...] / jnp.maximum(hgs_ref[...], 1.0) + bls_ref[...] + sr1_ref[...])
    xg = 0.5 * (dinv_g * (agg_ref[...] + ug1_ref[...]) + bgg_ref[...]
                + asg_ref[...] / jnp.maximum(hsg_ref[...], 1.0) + blg_ref[...] + gr1_ref[...])
    cols = lax.broadcasted_iota(i32, (_R, B), 1)
    oh_s = (bs_ref[0, 0][:, None] == cols).astype(f32)
    oh_g = (bg_ref[0, 0][:, None] == cols).astype(f32)
    cs = lax.dot_general(oh_s, xs, (((0,), (0,)), ((), ())), preferred_element_type=f32)
    cg = lax.dot_general(oh_g, xg, (((0,), (0,)), ((), ())), preferred_element_type=f32)

    @pl.when(i == 0)
    def _():
        ssum_ref[...] = cs
        gsum_ref[...] = cg

    @pl.when(i > 0)
    def _():
        ssum_ref[...] += cs
        gsum_ref[...] += cg

    @pl.when(i == (N // _R) - 1)
    def _():
        sm = ssum_ref[...] / jnp.maximum(bcs_ref[...], 1.0)
        gm = gsum_ref[...] / jnp.maximum(bcg_ref[...], 1.0)
        out_ref[...] = (jnp.dot(sm, wos_ref[...], preferred_element_type=f32)
                        + jnp.dot(gm, wog_ref[...], preferred_element_type=f32)
                        + dep_ref[...] * wod_ref[...] + bo_ref[...])


def _rowspec(w):
    return pl.BlockSpec((_R, w), lambda i: (i, 0))


def _fullspec(shape):
    return pl.BlockSpec(shape, lambda i: tuple(0 for _ in shape))


def _pad_edges(ei):
    src = jnp.concatenate([ei[0], jnp.zeros((EPAD - E,), i32)])
    dst = jnp.concatenate([ei[1], jnp.full((EPAD - E,), N, i32)])
    comb = jnp.stack([src.reshape(-1, K * CHUNK), dst.reshape(-1, K * CHUNK)],
                     axis=1).reshape(-1, CHUNK)
    return comb, dst.reshape(-1, CHUNK)


def kernel(x_state, x_goal, ei_ss, ei_gg, ei_sg, ei_gs, batch_state, batch_goal, depth,
           W_gcn_s_0, b_gcn_s_0, W_gcn_g_0, b_gcn_g_0, Wl_sg_0, bl_sg_0, Wr_sg_0,
           Wl_gs_0, bl_gs_0, Wr_gs_0,
           W_gcn_s_1, b_gcn_s_1, W_gcn_g_1, b_gcn_g_1, Wl_sg_1, bl_sg_1, Wr_sg_1,
           Wl_gs_1, bl_gs_1, Wr_gs_1, W_out, b_out):
    css, dss = _pad_edges(ei_ss.astype(i32))
    cgg, dgg = _pad_edges(ei_gg.astype(i32))
    csg, dsg = _pad_edges(ei_sg.astype(i32))
    cgs, dgs = _pad_edges(ei_gs.astype(i32))
    NB2 = BPADROWS * NTILE
    bs2 = jnp.concatenate([batch_state.astype(i32), jnp.full((NB2 * CHUNK - N,), B, i32)])
    bg2 = jnp.concatenate([batch_goal.astype(i32), jnp.full((NB2 * CHUNK - N,), B, i32)])

    z16 = jnp.zeros((ZROWS, 16), f32)
    o16 = jnp.ones((CHUNK, 16), f32)
    h_ss, h_gs, h_gg, h_sg, bc_s, bc_g = _hist_call()(dss, dgs, dgg, dsg, bs2, bg2, z16, o16)
    hss = h_ss[:, :1]
    hgs = h_gs[:, :1]
    hgg = h_gg[:, :1]
    hsg = h_sg[:, :1]

    wcs0 = jnp.concatenate([W_gcn_s_0, Wr_gs_0, Wl_sg_0], axis=1)
    wcg0 = jnp.concatenate([W_gcn_g_0, Wr_sg_0, Wl_gs_0], axis=1)
    grid = N // _R
    outH = [jax.ShapeDtypeStruct((N, H), f32)] * 6
    us0, sr0, sl0, ug0, gr0, gl0 = pl.pallas_call(
        _proj0_body,
        grid=(grid,),
        in_specs=[_rowspec(128), _rowspec(128), _rowspec(1), _rowspec(1),
                  _fullspec((128, 3 * H)), _fullspec((128, 3 * H))],
        out_specs=[_rowspec(H)] * 6,
        out_shape=outH,
    )(x_state, x_goal, hss, hgg, wcs0, wcg0)

    zH = jnp.zeros((NPAD, H), f32)
    a_ss0, a_gs0, a_gg0, a_sg0 = _agg_call()(
        us0, css, gl0, cgs, ug0, cgg, sl0, csg, zH)

    wcs1 = jnp.concatenate([W_gcn_s_1, Wr_gs_1, Wl_sg_1], axis=1)
    wcg1 = jnp.concatenate([W_gcn_g_1, Wr_sg_1, Wl_gs_1], axis=1)
    bgs0 = b_gcn_s_0.reshape(1, H)
    bls0 = bl_gs_0.reshape(1, H)
    bgg0 = b_gcn_g_0.reshape(1, H)
    blg0 = bl_sg_0.reshape(1, H)
    us1, sr1, sl1, ug1, gr1, gl1 = pl.pallas_call(
        _comb1_body,
        grid=(grid,),
        in_specs=[_rowspec(H), _rowspec(H), _rowspec(H), _rowspec(H), _rowspec(1), _rowspec(1),
                  _fullspec((1, H)), _fullspec((1, H)),
                  _rowspec(H), _rowspec(H), _rowspec(H), _rowspec(H), _rowspec(1), _rowspec(1),
                  _fullspec((1, H)), _fullspec((1, H)),
                  _fullspec((H, 3 * H)), _fullspec((H, 3 * H))],
        out_specs=[_rowspec(H)] * 6,
        out_shape=outH,
    )(a_ss0, a_gs0, us0, sr0, hss, hgs, bgs0, bls0,
      a_gg0, a_sg0, ug0, gr0, hgg, hsg, bgg0, blg0, wcs1, wcg1)

    a_ss1, a_gs1, a_gg1, a_sg1 = _agg_call()(
        us1, css, gl1, cgs, ug1, cgg, sl1, csg, zH)

    bs3 = batch_state.astype(i32).reshape(grid, 1, _R)
    bg3 = batch_goal.astype(i32).reshape(grid, 1, _R)
    bgs1 = b_gcn_s_1.reshape(1, H)
    bls1 = bl_gs_1.reshape(1, H)
    bgg1 = b_gcn_g_1.reshape(1, H)
    blg1 = bl_sg_1.reshape(1, H)
    _, _, out = pl.pallas_call(
        _final_body,
        grid=(grid,),
        in_specs=[_rowspec(H), _rowspec(H), _rowspec(H), _rowspec(H), _rowspec(1), _rowspec(1),
                  _fullspec((1, H)), _fullspec((1, H)),
                  _rowspec(H), _rowspec(H), _rowspec(H), _rowspec(H), _rowspec(1), _rowspec(1),
                  _fullspec((1, H)), _fullspec((1, H)),
                  pl.BlockSpec((1, 1, _R), lambda i: (i, 0, 0)),
                  pl.BlockSpec((1, 1, _R), lambda i: (i, 0, 0)),
                  _fullspec((B, 1)), _fullspec((B, 1)), _fullspec((B, 1)),
                  _fullspec((H, 1)), _fullspec((H, 1)), _fullspec((1, 1)), _fullspec((1, 1))],
        out_specs=[_fullspec((B, H)), _fullspec((B, H)), _fullspec((B, 1))],
        out_shape=[jax.ShapeDtypeStruct((B, H), f32), jax.ShapeDtypeStruct((B, H), f32),
                   jax.ShapeDtypeStruct((B, 1), f32)],
    )(a_ss1, a_gs1, us1, sr1, hss, hgs, bgs1, bls1,
      a_gg1, a_sg1, ug1, gr1, hgg, hsg, bgg1, blg1,
      bs3, bg3, bc_s[:, :1], bc_g[:, :1], depth.reshape(B, 1),
      W_out[:H], W_out[H:2 * H], W_out[2 * H:].reshape(1, 1), b_out.reshape(1, 1))
    return out[:, 0]


# speedup vs baseline: 24.9906x; 1.8819x over previous
import functools

import jax
import jax.numpy as jnp
from jax import lax
from jax.experimental import pallas as pl
from jax.experimental.pallas import tpu as pltpu
from jax.experimental.pallas import tpu_sc as plsc

N = 10000
NPAD = 10240
E = 320000
CHUNK = 128
K = 2
TCHUNKS = 160
GROUPS = TCHUNKS // K
EPAD = TCHUNKS * CHUNK * 16
H = 64
B = 256
NTILE = 16
ZROWS = NPAD // NTILE
ZB = 80
BPADROWS = 5
BACC = 272

_mesh = plsc.VectorSubcoreMesh(core_axis_name="c", subcore_axis_name="s")
f32 = jnp.float32
i32 = jnp.int32


def _hist_body(dss, dgs, dgg, dsg, bs2, bg2, zeros_h, ones_h,
               h_ss, h_gs, h_gg, h_sg, bc_s, bc_g,
               idxd, ones_v, zbuf, acc_a, acc_b, bacc, hs0, hs1):
    c = lax.axis_index("c")
    s = lax.axis_index("s")
    pltpu.sync_copy(zeros_h, zbuf)
    pltpu.sync_copy(ones_h, ones_v)
    pltpu.sync_copy(zbuf, acc_a.at[pl.ds(s * ZROWS, ZROWS)])
    pltpu.sync_copy(zbuf, acc_b.at[pl.ds(s * ZROWS, ZROWS)])

    @pl.when(s == 0)
    def _():
        pltpu.sync_copy(zbuf.at[pl.ds(0, BACC)], bacc)

    plsc.subcore_barrier()

    def run(dst2d, acc):
        base = s * GROUPS

        def stage(g, b):
            pltpu.sync_copy(dst2d.at[pl.ds((base + g) * K, K)], idxd.at[b])

        def fire(b, sem):
            for k in range(K):
                pltpu.async_copy(ones_v, acc.at[idxd.at[b, k]], sem, add=True)

        def drain(b, sem):
            for k in range(K):
                pltpu.make_async_copy(ones_v, acc.at[idxd.at[b, k]], sem).wait()

        stage(0, 0)

        def body(r, carry):
            for b, sem, nsem in ((0, hs0, hs1), (1, hs1, hs0)):
                g = 2 * r + b
                nb = 1 - b
                fire(b, sem)

                @pl.when(g >= 1)
                def _():
                    drain(nb, nsem)

                @pl.when(g + 1 < GROUPS)
                def _():
                    stage(g + 1, nb)
            return carry
        lax.fori_loop(0, GROUPS // 2, body, 0)
        drain(1, hs1)

    def runb(bat1d, acc, nrows):
        def chunk(j, carry):
            pltpu.sync_copy(bat1d.at[pl.ds((s * nrows + j) * CHUNK, CHUNK)], idxd.at[0, 0])
            pltpu.sync_copy(ones_v, acc.at[idxd.at[0, 0]], add=True)
            return carry
        lax.fori_loop(0, nrows, chunk, 0)

    @pl.when(c == 0)
    def _():
        run(dss, acc_a)
        run(dgs, acc_b)
        runb(bs2, bacc, BPADROWS)

    @pl.when(c == 1)
    def _():
        run(dgg, acc_a)
        run(dsg, acc_b)
        runb(bg2, bacc, BPADROWS)

    plsc.subcore_barrier()
    sl = pl.ds(s * ZROWS, ZROWS)

    @pl.when(c == 0)
    def _():
        pltpu.sync_copy(acc_a.at[sl], h_ss.at[sl])
        pltpu.sync_copy(acc_b.at[sl], h_gs.at[sl])

        @pl.when(s == 0)
        def _():
            pltpu.sync_copy(bacc.at[pl.ds(0, B)], bc_s)

    @pl.when(c == 1)
    def _():
        pltpu.sync_copy(acc_a.at[sl], h_gg.at[sl])
        pltpu.sync_copy(acc_b.at[sl], h_sg.at[sl])

        @pl.when(s == 0)
        def _():
            pltpu.sync_copy(bacc.at[pl.ds(0, B)], bc_g)


_hist_call = functools.partial(
    pl.kernel, _hist_body, mesh=_mesh,
    compiler_params=pltpu.CompilerParams(use_tc_tiling_on_sc=False),
    out_type=[jax.ShapeDtypeStruct((NPAD, 16), f32)] * 4 + [jax.ShapeDtypeStruct((B, 16), f32)] * 2,
    scratch_types=[
        pltpu.VMEM((2, K, CHUNK), i32),
        pltpu.VMEM((CHUNK, 16), f32),
        pltpu.VMEM((ZROWS, 16), f32),
        pltpu.VMEM_SHARED((NPAD, 16), f32),
        pltpu.VMEM_SHARED((NPAD, 16), f32),
        pltpu.VMEM_SHARED((BACC, 16), f32),
        pltpu.SemaphoreType.DMA,
        pltpu.SemaphoreType.DMA,
    ],
)


def _agg_body(tab_ss, css, tab_gs, cgs, tab_gg, cgg, tab_sg, csg, zeros_h,
              o_ss, o_gs, o_gg, o_sg,
              idxv, rows, acc, tabsh, is0, is1, is2, is3, gs0, gs1, ss0, ss1):
    c = lax.axis_index("c")
    s = lax.axis_index("s")
    sl = pl.ds(s * ZROWS, ZROWS)
    isems = (is0, is1, is2, is3)
    gsems = (gs0, gs1)
    ssems = (ss0, ss1)

    TS = N // NTILE
    def run(tab, comb2d, out):
        pltpu.sync_copy(zeros_h.at[sl], acc.at[sl])
        pltpu.sync_copy(tab.at[pl.ds(s * TS, TS)], tabsh.at[pl.ds(s * TS, TS)])
        plsc.subcore_barrier()
        base = s * GROUPS

        def istart(g, ib):
            pltpu.async_copy(comb2d.at[pl.ds((base + g) * 2 * K, 2 * K)],
                             idxv.at[ib], isems[ib])

        def iwait(g, ib):
            pltpu.make_async_copy(comb2d.at[pl.ds((base + g) * 2 * K, 2 * K)],
                                  idxv.at[ib], isems[ib]).wait()

        def gfire(ib, b):
            for k in range(K):
                pltpu.async_copy(tabsh.at[idxv.at[ib, k]], rows.at[b, k], gsems[b])

        def gdrain(ib, b):
            for k in range(K):
                pltpu.make_async_copy(tabsh.at[idxv.at[ib, k]], rows.at[b, k], gsems[b]).wait()

        def sfire(ib, b):
            for k in range(K):
                pltpu.async_copy(rows.at[b, k], acc.at[idxv.at[ib, K + k]], ssems[b], add=True)

        def sdrain(ib, b):
            for k in range(K):
                pltpu.make_async_copy(rows.at[b, k], acc.at[idxv.at[ib, K + k]], ssems[b]).wait()

        istart(0, 0)
        istart(1, 1)
        iwait(0, 0)
        gfire(0, 0)

        def body(r, carry):
            for u in range(4):
                g = 4 * r + u
                b = u % 2
                nb = 1 - b
                ib = u
                nib = (u + 1) % 4

                @pl.when(g + 2 < GROUPS)
                def _():
                    istart(g + 2, (u + 2) % 4)
                gdrain(ib, b)
                sfire(ib, b)

                @pl.when(g >= 1)
                def _():
                    sdrain((u - 1) % 4, nb)

                @pl.when(g + 1 < GROUPS)
                def _():
                    iwait(g + 1, nib)
                    gfire(nib, nb)
            return carry
        lax.fori_loop(0, GROUPS // 4, body, 0)
        sdrain(3, 1)
        plsc.subcore_barrier()
        pltpu.sync_copy(acc.at[sl], out.at[sl])

    @pl.when(c == 0)
    def _():
        run(tab_ss, css, o_ss)
        run(tab_gs, cgs, o_gs)

    @pl.when(c == 1)
    def _():
        run(tab_gg, cgg, o_gg)
        run(tab_sg, csg, o_sg)


_agg_call = functools.partial(
    pl.kernel, _agg_body, mesh=_mesh,
    compiler_params=pltpu.CompilerParams(use_tc_tiling_on_sc=False),
    out_type=[jax.ShapeDtypeStruct((NPAD, H), f32)] * 4,
    scratch_types=[
        pltpu.VMEM((4, 2 * K, CHUNK), i32),
        pltpu.VMEM((2, K, CHUNK, H), f32),
        pltpu.VMEM_SHARED((NPAD, H), f32),
        pltpu.VMEM_SHARED((N, H), f32),
    ] + [pltpu.SemaphoreType.DMA] * 8,
)


_R = 1000


def _proj0_body(xs_ref, xg_ref, hss_ref, hgg_ref, wcs_ref, wcg_ref,
                us_ref, sr_ref, sl_ref, ug_ref, gr_ref, gl_ref):
    ps = jnp.dot(xs_ref[...], wcs_ref[...], preferred_element_type=f32)
    pg = jnp.dot(xg_ref[...], wcg_ref[...], preferred_element_type=f32)
    dinv_s = lax.rsqrt(hss_ref[...] + 1.0)
    dinv_g = lax.rsqrt(hgg_ref[...] + 1.0)
    us_ref[...] = dinv_s * ps[:, :H]
    sr_ref[...] = ps[:, H:2 * H]
    sl_ref[...] = ps[:, 2 * H:]
    ug_ref[...] = dinv_g * pg[:, :H]
    gr_ref[...] = pg[:, H:2 * H]
    gl_ref[...] = pg[:, 2 * H:]


def _comb1_body(ass_ref, ags_ref, us0_ref, sr0_ref, hss_ref, hgs_ref, bgs_ref, bls_ref,
                agg_ref, asg_ref, ug0_ref, gr0_ref, hgg_ref, hsg_ref, bgg_ref, blg_ref,
                wcs_ref, wcg_ref,
                us_ref, sr_ref, sl_ref, ug_ref, gr_ref, gl_ref):
    dinv_s = lax.rsqrt(hss_ref[...] + 1.0)
    dinv_g = lax.rsqrt(hgg_ref[...] + 1.0)
    ns = 0.5 * (dinv_s * (ass_ref[...] + us0_ref[...]) + bgs_ref[...]
                + ags_ref[...] / jnp.maximum(hgs_ref[...], 1.0) + bls_ref[...] + sr0_ref[...])
    ng = 0.5 * (dinv_g * (agg_ref[...] + ug0_ref[...]) + bgg_ref[...]
                + asg_ref[...] / jnp.maximum(hsg_ref[...], 1.0) + blg_ref[...] + gr0_ref[...])
    ps = jnp.dot(ns, wcs_ref[...], preferred_element_type=f32)
    pg = jnp.dot(ng, wcg_ref[...], preferred_element_type=f32)
    us_ref[...] = dinv_s * ps[:, :H]
    sr_ref[...] = ps[:, H:2 * H]
    sl_ref[...] = ps[:, 2 * H:]
    ug_ref[...] = dinv_g * pg[:, :H]
    gr_ref[...] = pg[:, H:2 * H]
    gl_ref[...] = pg[:, 2 * H:]


def _final_body(ass_ref, ags_ref, us1_ref, sr1_ref, hss_ref, hgs_ref, bgs_ref, bls_ref,
                agg_ref, asg_ref, ug1_ref, gr1_ref, hgg_ref, hsg_ref, bgg_ref, blg_ref,
                bs_ref, bg_ref, bcs_ref, bcg_ref, dep_ref, wos_ref, wog_ref, wod_ref, bo_ref,
                ssum_ref, gsum_ref, out_ref):
    i = pl.program_id(0)
    dinv_s = lax.rsqrt(hss_ref[...] + 1.0)
    dinv_g = lax.rsqrt(hgg_ref[...] + 1.0)
    xs = 0.5 * (dinv_s * (ass_ref[...] + us1_ref[...]) + bgs_ref[...]
                + ags_ref[...] / jnp.maximum(hgs_ref[...], 1.0) + bls_ref[...] + sr1_ref[...])
    xg = 0.5 * (dinv_g * (agg_ref[...] + ug1_ref[...]) + bgg_ref[...]
                + asg_ref[...] / jnp.maximum(hsg_ref[...], 1.0) + blg_ref[...] + gr1_ref[...])
    cols = lax.broadcasted_iota(i32, (_R, B), 1)
    oh_s = (bs_ref[0, 0][:, None] == cols).astype(f32)
    oh_g = (bg_ref[0, 0][:, None] == cols).astype(f32)
    cs = lax.dot_general(oh_s, xs, (((0,), (0,)), ((), ())), preferred_element_type=f32)
    cg = lax.dot_general(oh_g, xg, (((0,), (0,)), ((), ())), preferred_element_type=f32)

    @pl.when(i == 0)
    def _():
        ssum_ref[...] = cs
        gsum_ref[...] = cg

    @pl.when(i > 0)
    def _():
        ssum_ref[...] += cs
        gsum_ref[...] += cg

    @pl.when(i == (N // _R) - 1)
    def _():
        sm = ssum_ref[...] / jnp.maximum(bcs_ref[...], 1.0)
        gm = gsum_ref[...] / jnp.maximum(bcg_ref[...], 1.0)
        out_ref[...] = (jnp.dot(sm, wos_ref[...], preferred_element_type=f32)
                        + jnp.dot(gm, wog_ref[...], preferred_element_type=f32)
                        + dep_ref[...] * wod_ref[...] + bo_ref[...])


def _rowspec(w):
    return pl.BlockSpec((_R, w), lambda i: (i, 0))


def _fullspec(shape):
    return pl.BlockSpec(shape, lambda i: tuple(0 for _ in shape))


def _pad_edges(ei):
    src = jnp.concatenate([ei[0], jnp.zeros((EPAD - E,), i32)])
    dst = jnp.concatenate([ei[1], jnp.full((EPAD - E,), N, i32)])
    comb = jnp.stack([src.reshape(-1, K * CHUNK), dst.reshape(-1, K * CHUNK)],
                     axis=1).reshape(-1, CHUNK)
    return comb, dst.reshape(-1, CHUNK)


def kernel(x_state, x_goal, ei_ss, ei_gg, ei_sg, ei_gs, batch_state, batch_goal, depth,
           W_gcn_s_0, b_gcn_s_0, W_gcn_g_0, b_gcn_g_0, Wl_sg_0, bl_sg_0, Wr_sg_0,
           Wl_gs_0, bl_gs_0, Wr_gs_0,
           W_gcn_s_1, b_gcn_s_1, W_gcn_g_1, b_gcn_g_1, Wl_sg_1, bl_sg_1, Wr_sg_1,
           Wl_gs_1, bl_gs_1, Wr_gs_1, W_out, b_out):
    css, dss = _pad_edges(ei_ss.astype(i32))
    cgg, dgg = _pad_edges(ei_gg.astype(i32))
    csg, dsg = _pad_edges(ei_sg.astype(i32))
    cgs, dgs = _pad_edges(ei_gs.astype(i32))
    NB2 = BPADROWS * NTILE
    bs2 = jnp.concatenate([batch_state.astype(i32), jnp.full((NB2 * CHUNK - N,), B, i32)])
    bg2 = jnp.concatenate([batch_goal.astype(i32), jnp.full((NB2 * CHUNK - N,), B, i32)])

    z16 = jnp.zeros((ZROWS, 16), f32)
    o16 = jnp.ones((CHUNK, 16), f32)
    h_ss, h_gs, h_gg, h_sg, bc_s, bc_g = _hist_call()(dss, dgs, dgg, dsg, bs2, bg2, z16, o16)
    hss = h_ss[:, :1]
    hgs = h_gs[:, :1]
    hgg = h_gg[:, :1]
    hsg = h_sg[:, :1]

    wcs0 = jnp.concatenate([W_gcn_s_0, Wr_gs_0, Wl_sg_0], axis=1)
    wcg0 = jnp.concatenate([W_gcn_g_0, Wr_sg_0, Wl_gs_0], axis=1)
    grid = N // _R
    outH = [jax.ShapeDtypeStruct((N, H), f32)] * 6
    us0, sr0, sl0, ug0, gr0, gl0 = pl.pallas_call(
        _proj0_body,
        grid=(grid,),
        in_specs=[_rowspec(128), _rowspec(128), _rowspec(1), _rowspec(1),
                  _fullspec((128, 3 * H)), _fullspec((128, 3 * H))],
        out_specs=[_rowspec(H)] * 6,
        out_shape=outH,
    )(x_state, x_goal, hss, hgg, wcs0, wcg0)

    zH = jnp.zeros((NPAD, H), f32)
    a_ss0, a_gs0, a_gg0, a_sg0 = _agg_call()(
        us0, css, gl0, cgs, ug0, cgg, sl0, csg, zH)

    wcs1 = jnp.concatenate([W_gcn_s_1, Wr_gs_1, Wl_sg_1], axis=1)
    wcg1 = jnp.concatenate([W_gcn_g_1, Wr_sg_1, Wl_gs_1], axis=1)
    bgs0 = b_gcn_s_0.reshape(1, H)
    bls0 = bl_gs_0.reshape(1, H)
    bgg0 = b_gcn_g_0.reshape(1, H)
    blg0 = bl_sg_0.reshape(1, H)
    us1, sr1, sl1, ug1, gr1, gl1 = pl.pallas_call(
        _comb1_body,
        grid=(grid,),
        in_specs=[_rowspec(H), _rowspec(H), _rowspec(H), _rowspec(H), _rowspec(1), _rowspec(1),
                  _fullspec((1, H)), _fullspec((1, H)),
                  _rowspec(H), _rowspec(H), _rowspec(H), _rowspec(H), _rowspec(1), _rowspec(1),
                  _fullspec((1, H)), _fullspec((1, H)),
                  _fullspec((H, 3 * H)), _fullspec((H, 3 * H))],
        out_specs=[_rowspec(H)] * 6,
        out_shape=outH,
    )(a_ss0, a_gs0, us0, sr0, hss, hgs, bgs0, bls0,
      a_gg0, a_sg0, ug0, gr0, hgg, hsg, bgg0, blg0, wcs1, wcg1)

    a_ss1, a_gs1, a_gg1, a_sg1 = _agg_call()(
        us1, css, gl1, cgs, ug1, cgg, sl1, csg, zH)

    bs3 = batch_state.astype(i32).reshape(grid, 1, _R)
    bg3 = batch_goal.astype(i32).reshape(grid, 1, _R)
    bgs1 = b_gcn_s_1.reshape(1, H)
    bls1 = bl_gs_1.reshape(1, H)
    bgg1 = b_gcn_g_1.reshape(1, H)
    blg1 = bl_sg_1.reshape(1, H)
    _, _, out = pl.pallas_call(
        _final_body,
        grid=(grid,),
        in_specs=[_rowspec(H), _rowspec(H), _rowspec(H), _rowspec(H), _rowspec(1), _rowspec(1),
                  _fullspec((1, H)), _fullspec((1, H)),
                  _rowspec(H), _rowspec(H), _rowspec(H), _rowspec(H), _rowspec(1), _rowspec(1),
                  _fullspec((1, H)), _fullspec((1, H)),
                  pl.BlockSpec((1, 1, _R), lambda i: (i, 0, 0)),
                  pl.BlockSpec((1, 1, _R), lambda i: (i, 0, 0)),
                  _fullspec((B, 1)), _fullspec((B, 1)), _fullspec((B, 1)),
                  _fullspec((H, 1)), _fullspec((H, 1)), _fullspec((1, 1)), _fullspec((1, 1))],
        out_specs=[_fullspec((B, H)), _fullspec((B, H)), _fullspec((B, 1))],
        out_shape=[jax.ShapeDtypeStruct((B, H), f32), jax.ShapeDtypeStruct((B, H), f32),
                   jax.ShapeDtypeStruct((B, 1), f32)],
    )(a_ss1, a_gs1, us1, sr1, hss, hgs, bgs1, bls1,
      a_gg1, a_sg1, ug1, gr1, hgg, hsg, bgg1, blg1,
      bs3, bg3, bc_s[:, :1], bc_g[:, :1], depth.reshape(B, 1),
      W_out[:H], W_out[H:2 * H], W_out[2 * H:].reshape(1, 1), b_out.reshape(1, 1))
    return out[:, 0]


# speedup vs baseline: 26.4100x; 1.0568x over previous
import functools

import jax
import jax.numpy as jnp
from jax import lax
from jax.experimental import pallas as pl
from jax.experimental.pallas import tpu as pltpu
from jax.experimental.pallas import tpu_sc as plsc

N = 10000
NPAD = 10240
E = 320000
CHUNK = 128
K = 2
TCHUNKS = 160
GROUPS = TCHUNKS // K
HK = 4
HGROUPS = TCHUNKS // HK
EPAD = TCHUNKS * CHUNK * 16
H = 64
B = 256
NTILE = 16
ZROWS = NPAD // NTILE
ZB = 80
BPADROWS = 5
BACC = 272

_mesh = plsc.VectorSubcoreMesh(core_axis_name="c", subcore_axis_name="s")
f32 = jnp.float32
i32 = jnp.int32


def _hist_body(dss, dgs, dgg, dsg, bs2, bg2, zeros_h, ones_h,
               h_ss, h_gs, h_gg, h_sg, bc_s, bc_g,
               idxd, ones_v, zbuf, acc_a, acc_b, bacc, hs0, hs1):
    c = lax.axis_index("c")
    s = lax.axis_index("s")
    pltpu.sync_copy(zeros_h, zbuf)
    pltpu.sync_copy(ones_h, ones_v)
    pltpu.sync_copy(zbuf, acc_a.at[pl.ds(s * ZROWS, ZROWS)])
    pltpu.sync_copy(zbuf, acc_b.at[pl.ds(s * ZROWS, ZROWS)])

    @pl.when(s == 0)
    def _():
        pltpu.sync_copy(zbuf.at[pl.ds(0, BACC)], bacc)

    plsc.subcore_barrier()

    def run(dst2d, acc):
        base = s * HGROUPS

        def stage(g, b):
            pltpu.sync_copy(dst2d.at[pl.ds((base + g) * HK, HK)], idxd.at[b])

        def fire(b, sem):
            for k in range(HK):
                pltpu.async_copy(ones_v, acc.at[idxd.at[b, k]], sem, add=True)

        def drain(b, sem):
            for k in range(HK):
                pltpu.make_async_copy(ones_v, acc.at[idxd.at[b, k]], sem).wait()

        stage(0, 0)

        def body(r, carry):
            for b, sem, nsem in ((0, hs0, hs1), (1, hs1, hs0)):
                g = 2 * r + b
                nb = 1 - b
                fire(b, sem)

                @pl.when(g >= 1)
                def _():
                    drain(nb, nsem)

                @pl.when(g + 1 < HGROUPS)
                def _():
                    stage(g + 1, nb)
            return carry
        lax.fori_loop(0, HGROUPS // 2, body, 0)
        drain(1, hs1)

    def runb(bat1d, acc, nrows):
        def chunk(j, carry):
            pltpu.sync_copy(bat1d.at[pl.ds((s * nrows + j) * CHUNK, CHUNK)], idxd.at[0, 0])
            pltpu.sync_copy(ones_v, acc.at[idxd.at[0, 0]], add=True)
            return carry
        lax.fori_loop(0, nrows, chunk, 0)

    @pl.when(c == 0)
    def _():
        run(dss, acc_a)
        run(dgs, acc_b)
        runb(bs2, bacc, BPADROWS)

    @pl.when(c == 1)
    def _():
        run(dgg, acc_a)
        run(dsg, acc_b)
        runb(bg2, bacc, BPADROWS)

    plsc.subcore_barrier()
    sl = pl.ds(s * ZROWS, ZROWS)

    @pl.when(c == 0)
    def _():
        pltpu.sync_copy(acc_a.at[sl], h_ss.at[sl])
        pltpu.sync_copy(acc_b.at[sl], h_gs.at[sl])

        @pl.when(s == 0)
        def _():
            pltpu.sync_copy(bacc.at[pl.ds(0, B)], bc_s)

    @pl.when(c == 1)
    def _():
        pltpu.sync_copy(acc_a.at[sl], h_gg.at[sl])
        pltpu.sync_copy(acc_b.at[sl], h_sg.at[sl])

        @pl.when(s == 0)
        def _():
            pltpu.sync_copy(bacc.at[pl.ds(0, B)], bc_g)


_hist_call = functools.partial(
    pl.kernel, _hist_body, mesh=_mesh,
    compiler_params=pltpu.CompilerParams(use_tc_tiling_on_sc=False),
    out_type=[jax.ShapeDtypeStruct((NPAD, 16), f32)] * 4 + [jax.ShapeDtypeStruct((B, 16), f32)] * 2,
    scratch_types=[
        pltpu.VMEM((2, HK, CHUNK), i32),
        pltpu.VMEM((CHUNK, 16), f32),
        pltpu.VMEM((ZROWS, 16), f32),
        pltpu.VMEM_SHARED((NPAD, 16), f32),
        pltpu.VMEM_SHARED((NPAD, 16), f32),
        pltpu.VMEM_SHARED((BACC, 16), f32),
        pltpu.SemaphoreType.DMA,
        pltpu.SemaphoreType.DMA,
    ],
)


def _agg_body(tab_ss, css, tab_gs, cgs, tab_gg, cgg, tab_sg, csg, zeros_h,
              o_ss, o_gs, o_gg, o_sg,
              idxv, rows, acc, tabsh, is0, is1, is2, is3, gs0, gs1, ss0, ss1):
    c = lax.axis_index("c")
    s = lax.axis_index("s")
    sl = pl.ds(s * ZROWS, ZROWS)
    isems = (is0, is1, is2, is3)
    gsems = (gs0, gs1)
    ssems = (ss0, ss1)

    TS = N // NTILE
    def run(tab, comb2d, out):
        pltpu.sync_copy(zeros_h.at[sl], acc.at[sl])
        pltpu.sync_copy(tab.at[pl.ds(s * TS, TS)], tabsh.at[pl.ds(s * TS, TS)])
        plsc.subcore_barrier()
        base = s * GROUPS

        def istart(g, ib):
            pltpu.async_copy(comb2d.at[pl.ds((base + g) * 2 * K, 2 * K)],
                             idxv.at[ib], isems[ib])

        def iwait(g, ib):
            pltpu.make_async_copy(comb2d.at[pl.ds((base + g) * 2 * K, 2 * K)],
                                  idxv.at[ib], isems[ib]).wait()

        def gfire(ib, b):
            for k in range(K):
                pltpu.async_copy(tabsh.at[idxv.at[ib, k]], rows.at[b, k], gsems[b])

        def gdrain(ib, b):
            for k in range(K):
                pltpu.make_async_copy(tabsh.at[idxv.at[ib, k]], rows.at[b, k], gsems[b]).wait()

        def sfire(ib, b):
            for k in range(K):
                pltpu.async_copy(rows.at[b, k], acc.at[idxv.at[ib, K + k]], ssems[b], add=True)

        def sdrain(ib, b):
            for k in range(K):
                pltpu.make_async_copy(rows.at[b, k], acc.at[idxv.at[ib, K + k]], ssems[b]).wait()

        istart(0, 0)
        istart(1, 1)
        iwait(0, 0)
        gfire(0, 0)

        def body(r, carry):
            for u in range(4):
                g = 4 * r + u
                b = u % 2
                nb = 1 - b
                ib = u
                nib = (u + 1) % 4

                @pl.when(g + 2 < GROUPS)
                def _():
                    istart(g + 2, (u + 2) % 4)
                gdrain(ib, b)
                sfire(ib, b)

                @pl.when(g >= 1)
                def _():
                    sdrain((u - 1) % 4, nb)

                @pl.when(g + 1 < GROUPS)
                def _():
                    iwait(g + 1, nib)
                    gfire(nib, nb)
            return carry
        lax.fori_loop(0, GROUPS // 4, body, 0)
        sdrain(3, 1)
        plsc.subcore_barrier()
        pltpu.sync_copy(acc.at[sl], out.at[sl])

    @pl.when(c == 0)
    def _():
        run(tab_ss, css, o_ss)
        run(tab_gs, cgs, o_gs)

    @pl.when(c == 1)
    def _():
        run(tab_gg, cgg, o_gg)
        run(tab_sg, csg, o_sg)


_agg_call = functools.partial(
    pl.kernel, _agg_body, mesh=_mesh,
    compiler_params=pltpu.CompilerParams(use_tc_tiling_on_sc=False),
    out_type=[jax.ShapeDtypeStruct((NPAD, H), f32)] * 4,
    scratch_types=[
        pltpu.VMEM((4, 2 * K, CHUNK), i32),
        pltpu.VMEM((2, K, CHUNK, H), f32),
        pltpu.VMEM_SHARED((NPAD, H), f32),
        pltpu.VMEM_SHARED((N, H), f32),
    ] + [pltpu.SemaphoreType.DMA] * 8,
)


_R = 1000


def _proj0_body(xs_ref, xg_ref, wcs_ref, wcg_ref,
                p0s_ref, sr_ref, sl_ref, p0g_ref, gr_ref, gl_ref):
    ps = jnp.dot(xs_ref[...], wcs_ref[...], preferred_element_type=f32)
    pg = jnp.dot(xg_ref[...], wcg_ref[...], preferred_element_type=f32)
    p0s_ref[...] = ps[:, :H]
    sr_ref[...] = ps[:, H:2 * H]
    sl_ref[...] = ps[:, 2 * H:]
    p0g_ref[...] = pg[:, :H]
    gr_ref[...] = pg[:, H:2 * H]
    gl_ref[...] = pg[:, 2 * H:]


def _scale_body(p0s_ref, p0g_ref, hss_ref, hgg_ref, us_ref, ug_ref):
    us_ref[...] = lax.rsqrt(hss_ref[...] + 1.0) * p0s_ref[...]
    ug_ref[...] = lax.rsqrt(hgg_ref[...] + 1.0) * p0g_ref[...]


def _comb1_body(ass_ref, ags_ref, us0_ref, sr0_ref, hss_ref, hgs_ref, bgs_ref, bls_ref,
                agg_ref, asg_ref, ug0_ref, gr0_ref, hgg_ref, hsg_ref, bgg_ref, blg_ref,
                wcs_ref, wcg_ref,
                us_ref, sr_ref, sl_ref, ug_ref, gr_ref, gl_ref):
    dinv_s = lax.rsqrt(hss_ref[...] + 1.0)
    dinv_g = lax.rsqrt(hgg_ref[...] + 1.0)
    ns = 0.5 * (dinv_s * (ass_ref[...] + us0_ref[...]) + bgs_ref[...]
                + ags_ref[...] / jnp.maximum(hgs_ref[...], 1.0) + bls_ref[...] + sr0_ref[...])
    ng = 0.5 * (dinv_g * (agg_ref[...] + ug0_ref[...]) + bgg_ref[...]
                + asg_ref[...] / jnp.maximum(hsg_ref[...], 1.0) + blg_ref[...] + gr0_ref[...])
    ps = jnp.dot(ns, wcs_ref[...], preferred_element_type=f32)
    pg = jnp.dot(ng, wcg_ref[...], preferred_element_type=f32)
    us_ref[...] = dinv_s * ps[:, :H]
    sr_ref[...] = ps[:, H:2 * H]
    sl_ref[...] = ps[:, 2 * H:]
    ug_ref[...] = dinv_g * pg[:, :H]
    gr_ref[...] = pg[:, H:2 * H]
    gl_ref[...] = pg[:, 2 * H:]


def _final_body(ass_ref, ags_ref, us1_ref, sr1_ref, hss_ref, hgs_ref, bgs_ref, bls_ref,
                agg_ref, asg_ref, ug1_ref, gr1_ref, hgg_ref, hsg_ref, bgg_ref, blg_ref,
                bs_ref, bg_ref, bcs_ref, bcg_ref, dep_ref, wos_ref, wog_ref, wod_ref, bo_ref,
                ssum_ref, gsum_ref, out_ref):
    i = pl.program_id(0)
    dinv_s = lax.rsqrt(hss_ref[...] + 1.0)
    dinv_g = lax.rsqrt(hgg_ref[...] + 1.0)
    xs = 0.5 * (dinv_s * (ass_ref[...] + us1_ref[...]) + bgs_ref[...]
                + ags_ref[...] / jnp.maximum(hgs_ref[...], 1.0) + bls_ref[...] + sr1_ref[...])
    xg = 0.5 * (dinv_g * (agg_ref[...] + ug1_ref[...]) + bgg_ref[...]
                + asg_ref[...] / jnp.maximum(hsg_ref[...], 1.0) + blg_ref[...] + gr1_ref[...])
    cols = lax.broadcasted_iota(i32, (_R, B), 1)
    oh_s = (bs_ref[0, 0][:, None] == cols).astype(f32)
    oh_g = (bg_ref[0, 0][:, None] == cols).astype(f32)
    cs = lax.dot_general(oh_s, xs, (((0,), (0,)), ((), ())), preferred_element_type=f32)
    cg = lax.dot_general(oh_g, xg, (((0,), (0,)), ((), ())), preferred_element_type=f32)

    @pl.when(i == 0)
    def _():
        ssum_ref[...] = cs
        gsum_ref[...] = cg

    @pl.when(i > 0)
    def _():
        ssum_ref[...] += cs
        gsum_ref[...] += cg

    @pl.when(i == (N // _R) - 1)
    def _():
        sm = ssum_ref[...] / jnp.maximum(bcs_ref[...], 1.0)
        gm = gsum_ref[...] / jnp.maximum(bcg_ref[...], 1.0)
        out_ref[...] = (jnp.dot(sm, wos_ref[...], preferred_element_type=f32)
                        + jnp.dot(gm, wog_ref[...], preferred_element_type=f32)
                        + dep_ref[...] * wod_ref[...] + bo_ref[...])


def _rowspec(w):
    return pl.BlockSpec((_R, w), lambda i: (i, 0))


def _fullspec(shape):
    return pl.BlockSpec(shape, lambda i: tuple(0 for _ in shape))


def _pad_edges(ei):
    src = jnp.concatenate([ei[0], jnp.zeros((EPAD - E,), i32)])
    dst = jnp.concatenate([ei[1], jnp.full((EPAD - E,), N, i32)])
    comb = jnp.stack([src.reshape(-1, K * CHUNK), dst.reshape(-1, K * CHUNK)],
                     axis=1).reshape(-1, CHUNK)
    return comb, dst.reshape(-1, CHUNK)


def kernel(x_state, x_goal, ei_ss, ei_gg, ei_sg, ei_gs, batch_state, batch_goal, depth,
           W_gcn_s_0, b_gcn_s_0, W_gcn_g_0, b_gcn_g_0, Wl_sg_0, bl_sg_0, Wr_sg_0,
           Wl_gs_0, bl_gs_0, Wr_gs_0,
           W_gcn_s_1, b_gcn_s_1, W_gcn_g_1, b_gcn_g_1, Wl_sg_1, bl_sg_1, Wr_sg_1,
           Wl_gs_1, bl_gs_1, Wr_gs_1, W_out, b_out):
    css, dss = _pad_edges(ei_ss.astype(i32))
    cgg, dgg = _pad_edges(ei_gg.astype(i32))
    csg, dsg = _pad_edges(ei_sg.astype(i32))
    cgs, dgs = _pad_edges(ei_gs.astype(i32))
    NB2 = BPADROWS * NTILE
    bs2 = jnp.concatenate([batch_state.astype(i32), jnp.full((NB2 * CHUNK - N,), B, i32)])
    bg2 = jnp.concatenate([batch_goal.astype(i32), jnp.full((NB2 * CHUNK - N,), B, i32)])

    z16 = jnp.zeros((ZROWS, 16), f32)
    o16 = jnp.ones((CHUNK, 16), f32)
    h_ss, h_gs, h_gg, h_sg, bc_s, bc_g = _hist_call()(dss, dgs, dgg, dsg, bs2, bg2, z16, o16)
    hss = h_ss[:, :1]
    hgs = h_gs[:, :1]
    hgg = h_gg[:, :1]
    hsg = h_sg[:, :1]

    wcs0 = jnp.concatenate([W_gcn_s_0, Wr_gs_0, Wl_sg_0], axis=1)
    wcg0 = jnp.concatenate([W_gcn_g_0, Wr_sg_0, Wl_gs_0], axis=1)
    grid = N // _R
    outH = [jax.ShapeDtypeStruct((N, H), f32)] * 6
    p0s, sr0, sl0, p0g, gr0, gl0 = pl.pallas_call(
        _proj0_body,
        grid=(grid,),
        in_specs=[_rowspec(128), _rowspec(128),
                  _fullspec((128, 3 * H)), _fullspec((128, 3 * H))],
        out_specs=[_rowspec(H)] * 6,
        out_shape=outH,
    )(x_state, x_goal, wcs0, wcg0)
    us0, ug0 = pl.pallas_call(
        _scale_body,
        grid=(grid,),
        in_specs=[_rowspec(H), _rowspec(H), _rowspec(1), _rowspec(1)],
        out_specs=[_rowspec(H)] * 2,
        out_shape=outH[:2],
    )(p0s, p0g, hss, hgg)

    zH = jnp.zeros((NPAD, H), f32)
    a_ss0, a_gs0, a_gg0, a_sg0 = _agg_call()(
        us0, css, gl0, cgs, ug0, cgg, sl0, csg, zH)

    wcs1 = jnp.concatenate([W_gcn_s_1, Wr_gs_1, Wl_sg_1], axis=1)
    wcg1 = jnp.concatenate([W_gcn_g_1, Wr_sg_1, Wl_gs_1], axis=1)
    bgs0 = b_gcn_s_0.reshape(1, H)
    bls0 = bl_gs_0.reshape(1, H)
    bgg0 = b_gcn_g_0.reshape(1, H)
    blg0 = bl_sg_0.reshape(1, H)
    us1, sr1, sl1, ug1, gr1, gl1 = pl.pallas_call(
        _comb1_body,
        grid=(grid,),
        in_specs=[_rowspec(H), _rowspec(H), _rowspec(H), _rowspec(H), _rowspec(1), _rowspec(1),
                  _fullspec((1, H)), _fullspec((1, H)),
                  _rowspec(H), _rowspec(H), _rowspec(H), _rowspec(H), _rowspec(1), _rowspec(1),
                  _fullspec((1, H)), _fullspec((1, H)),
                  _fullspec((H, 3 * H)), _fullspec((H, 3 * H))],
        out_specs=[_rowspec(H)] * 6,
        out_shape=outH,
    )(a_ss0, a_gs0, us0, sr0, hss, hgs, bgs0, bls0,
      a_gg0, a_sg0, ug0, gr0, hgg, hsg, bgg0, blg0, wcs1, wcg1)

    a_ss1, a_gs1, a_gg1, a_sg1 = _agg_call()(
        us1, css, gl1, cgs, ug1, cgg, sl1, csg, zH)

    bs3 = batch_state.astype(i32).reshape(grid, 1, _R)
    bg3 = batch_goal.astype(i32).reshape(grid, 1, _R)
    bgs1 = b_gcn_s_1.reshape(1, H)
    bls1 = bl_gs_1.reshape(1, H)
    bgg1 = b_gcn_g_1.reshape(1, H)
    blg1 = bl_sg_1.reshape(1, H)
    _, _, out = pl.pallas_call(
        _final_body,
        grid=(grid,),
        in_specs=[_rowspec(H), _rowspec(H), _rowspec(H), _rowspec(H), _rowspec(1), _rowspec(1),
                  _fullspec((1, H)), _fullspec((1, H)),
                  _rowspec(H), _rowspec(H), _rowspec(H), _rowspec(H), _rowspec(1), _rowspec(1),
                  _fullspec((1, H)), _fullspec((1, H)),
                  pl.BlockSpec((1, 1, _R), lambda i: (i, 0, 0)),
                  pl.BlockSpec((1, 1, _R), lambda i: (i, 0, 0)),
                  _fullspec((B, 1)), _fullspec((B, 1)), _fullspec((B, 1)),
                  _fullspec((H, 1)), _fullspec((H, 1)), _fullspec((1, 1)), _fullspec((1, 1))],
        out_specs=[_fullspec((B, H)), _fullspec((B, H)), _fullspec((B, 1))],
        out_shape=[jax.ShapeDtypeStruct((B, H), f32), jax.ShapeDtypeStruct((B, H), f32),
                   jax.ShapeDtypeStruct((B, 1), f32)],
    )(a_ss1, a_gs1, us1, sr1, hss, hgs, bgs1, bls1,
      a_gg1, a_sg1, ug1, gr1, hgg, hsg, bgg1, blg1,
      bs3, bg3, bc_s[:, :1], bc_g[:, :1], depth.reshape(B, 1),
      W_out[:H], W_out[H:2 * H], W_out[2 * H:].reshape(1, 1), b_out.reshape(1, 1))
    return out[:, 0]
